# Initial kernel scaffold; baseline (speedup 1.0000x reference)
import jax
import jax.numpy as jnp
from jax.experimental import pallas as pl


def kernel(edge_index, item, alias, mask, tar, embedding, W_gc_0, b_gc_0, W_gc_1, b_gc_1, W_concat, b_concat, nasr_w2, nasr_w3, nasr_v, nasr_b):
    raise NotImplementedError("write your pallas kernel here")


# speedup vs baseline: 1.0601x; 1.0601x over previous
import functools

import jax
import jax.numpy as jnp
from jax import lax
from jax.experimental import pallas as pl
from jax.experimental.pallas import tpu as pltpu
from jax.experimental.pallas import tpu_sc as plsc

N = 50000
H = 100
HP = 128
B = 256
T = 50
E = 800000
V = N - 1
L2C = 1e-05

RT = 51200
RB = 3200
CB = 2048


def _leaky(x):
    return jnp.where(x >= 0, x, 0.2 * x)


def _mm_act_body(x_ref, w_ref, b_ref, o_ref):
    o_ref[...] = _leaky(
        jnp.dot(x_ref[...], w_ref[...], preferred_element_type=jnp.float32, precision=lax.Precision.HIGHEST)
        + b_ref[...])


def _mm_act(x, w, b):
    return pl.pallas_call(
        _mm_act_body,
        grid=(RT // RB,),
        in_specs=[
            pl.BlockSpec((RB, HP), lambda i: (i, 0)),
            pl.BlockSpec((HP, HP), lambda i: (0, 0)),
            pl.BlockSpec((1, HP), lambda i: (0, 0)),
        ],
        out_specs=pl.BlockSpec((RB, HP), lambda i: (i, 0)),
        out_shape=jax.ShapeDtypeStruct((RT, HP), jnp.float32),
    )(x, w, b)


def _final_body(agg2_ref, e0_ref, e1_ref, w1_ref, b1_ref,
                wc0_ref, wc1_ref, wc2_ref, bc_ref, o_ref, ssq_ref):
    i = pl.program_id(0)
    e0 = e0_ref[...]
    e2 = _leaky(
        jnp.dot(agg2_ref[...], w1_ref[...], preferred_element_type=jnp.float32, precision=lax.Precision.HIGHEST)
        + b1_ref[...])
    f = (jnp.dot(e0, wc0_ref[...], preferred_element_type=jnp.float32, precision=lax.Precision.HIGHEST)
         + jnp.dot(e1_ref[...], wc1_ref[...], preferred_element_type=jnp.float32, precision=lax.Precision.HIGHEST)
         + jnp.dot(e2, wc2_ref[...], preferred_element_type=jnp.float32, precision=lax.Precision.HIGHEST)
         + bc_ref[...])
    o_ref[...] = _leaky(f)
    part = jnp.sum(e0 * e0).reshape(1, 1)

    @pl.when(i == 0)
    def _():
        ssq_ref[...] = part

    @pl.when(i > 0)
    def _():
        ssq_ref[...] += part


def _final_stage(agg2p, e0p, e1p, w1, b1, wc0, wc1, wc2, bc):
    return pl.pallas_call(
        _final_body,
        grid=(RT // RB,),
        in_specs=[
            pl.BlockSpec((RB, HP), lambda i: (i, 0)),
            pl.BlockSpec((RB, HP), lambda i: (i, 0)),
            pl.BlockSpec((RB, HP), lambda i: (i, 0)),
        ] + [pl.BlockSpec((HP, HP), lambda i: (0, 0))] * 1
          + [pl.BlockSpec((1, HP), lambda i: (0, 0))]
          + [pl.BlockSpec((HP, HP), lambda i: (0, 0))] * 3
          + [pl.BlockSpec((1, HP), lambda i: (0, 0))],
        out_specs=[
            pl.BlockSpec((RB, HP), lambda i: (i, 0)),
            pl.BlockSpec((1, 1), lambda i: (0, 0)),
        ],
        out_shape=[
            jax.ShapeDtypeStruct((RT, HP), jnp.float32),
            jax.ShapeDtypeStruct((1, 1), jnp.float32),
        ],
    )(agg2p, e0p, e1p, w1, b1, wc0, wc1, wc2, bc)


_SB = 32
_XB = _SB * T


def _session_body(x_ref, last_ref, valid_ref, maskf_ref, et_ref,
                  w2_ref, w3_ref, v_ref, nb_ref, ma_ref):
    x = x_ref[...]
    lasth = last_ref[...]
    ri = lax.broadcasted_iota(jnp.int32, (_XB, _SB), 0) // T
    ci = lax.broadcasted_iota(jnp.int32, (_XB, _SB), 1)
    sel = (ri == ci).astype(jnp.float32)
    lastb = jnp.dot(sel, lasth, preferred_element_type=jnp.float32, precision=lax.Precision.HIGHEST)
    seq = jnp.dot(x, w2_ref[...], preferred_element_type=jnp.float32, precision=lax.Precision.HIGHEST)
    m = jax.nn.sigmoid(lastb + seq + nb_ref[...])
    coef = jnp.dot(m, v_ref[...].T, preferred_element_type=jnp.float32, precision=lax.Precision.HIGHEST)
    coef = coef * maskf_ref[...]
    glb = lax.dot_general(sel, coef * x, (((0,), (0,)), ((), ())),
                          preferred_element_type=jnp.float32, precision=lax.Precision.HIGHEST)
    vald = valid_ref[...]
    cnt = jnp.clip(jnp.sum(vald, axis=1, keepdims=True), 1.0, None)
    last_emb = jnp.dot(vald, et_ref[...], preferred_element_type=jnp.float32, precision=lax.Precision.HIGHEST) / cnt
    gate = glb + jnp.dot(last_emb, w3_ref[...], preferred_element_type=jnp.float32, precision=lax.Precision.HIGHEST)
    ma_ref[...] = gate * lasth


def _session_stage(seq_hp, last_h, valid, maskf, etp, w2, w3, vp, nbp):
    return pl.pallas_call(
        _session_body,
        grid=(B // _SB,),
        in_specs=[
            pl.BlockSpec((_XB, HP), lambda i: (i, 0)),
            pl.BlockSpec((_SB, HP), lambda i: (i, 0)),
            pl.BlockSpec((_SB, T), lambda i: (i, 0)),
            pl.BlockSpec((_XB, 1), lambda i: (i, 0)),
            pl.BlockSpec((T, HP), lambda i: (0, 0)),
            pl.BlockSpec((HP, HP), lambda i: (0, 0)),
            pl.BlockSpec((HP, HP), lambda i: (0, 0)),
            pl.BlockSpec((1, HP), lambda i: (0, 0)),
            pl.BlockSpec((1, HP), lambda i: (0, 0)),
        ],
        out_specs=pl.BlockSpec((_SB, HP), lambda i: (i, 0)),
        out_shape=jax.ShapeDtypeStruct((B, HP), jnp.float32),
    )(seq_hp, last_h, valid, maskf, etp, w2, w3, vp, nbp)


def _logits_body(ma_ref, bt_ref, lab_ref, o_ref, mx_ref, sm_ref, lg_ref):
    i = pl.program_id(0)
    bt = bt_ref[...]
    lg = lax.dot_general(ma_ref[...], bt, (((1,), (1,)), ((), ())),
                         preferred_element_type=jnp.float32, precision=lax.Precision.HIGHEST)
    col = i * CB + lax.broadcasted_iota(jnp.int32, (B, CB), 1)
    okc = col < V
    lgm = jnp.where(okc, lg, -1e30)
    o_ref[...] = lg
    bm = jnp.max(lgm, axis=1, keepdims=True)
    bs = jnp.sum(jnp.exp(lgm - bm), axis=1, keepdims=True)
    labs = lab_ref[...]
    hit = jnp.where((col == labs) & okc, lg, 0.0)
    lab_part = jnp.sum(hit, axis=1, keepdims=True)

    @pl.when(i == 0)
    def _():
        mx_ref[...] = bm
        sm_ref[...] = bs
        lg_ref[...] = lab_part

    @pl.when(i > 0)
    def _():
        m_old = mx_ref[...]
        m_new = jnp.maximum(m_old, bm)
        sm_ref[...] = (sm_ref[...] * jnp.exp(m_old - m_new)
                       + bs * jnp.exp(bm - m_new))
        mx_ref[...] = m_new
        lg_ref[...] += lab_part


def _logits_stage(map_, btabp, labels):
    nblk = pl.cdiv(V, CB)
    return pl.pallas_call(
        _logits_body,
        grid=(nblk,),
        in_specs=[
            pl.BlockSpec((B, HP), lambda i: (0, 0)),
            pl.BlockSpec((CB, HP), lambda i: (i, 0)),
            pl.BlockSpec((B, 1), lambda i: (0, 0)),
        ],
        out_specs=[
            pl.BlockSpec((B, CB), lambda i: (0, i)),
            pl.BlockSpec((B, 1), lambda i: (0, 0)),
            pl.BlockSpec((B, 1), lambda i: (0, 0)),
            pl.BlockSpec((B, 1), lambda i: (0, 0)),
        ],
        out_shape=[
            jax.ShapeDtypeStruct((B, V), jnp.float32),
            jax.ShapeDtypeStruct((B, 1), jnp.float32),
            jax.ShapeDtypeStruct((B, 1), jnp.float32),
            jax.ShapeDtypeStruct((B, 1), jnp.float32),
        ],
    )(map_, btabp, labels)


def _scatter_layer(ep, src, dst):
    return jnp.zeros((RT, HP), jnp.float32).at[dst].add(ep[src])


def _session_gather(finalp, gidx):
    return finalp[gidx]


def _pad_w(w):
    return jnp.pad(w, ((0, HP - w.shape[0]), (0, HP - w.shape[1])))


def _pad_b(b):
    return jnp.pad(b.reshape(1, -1), ((0, 0), (0, HP - b.shape[-1])))


@jax.jit
def kernel(edge_index, item, alias, mask, tar, embedding,
           W_gc_0, b_gc_0, W_gc_1, b_gc_1, W_concat, b_concat,
           nasr_w2, nasr_w3, nasr_v, nasr_b):
    src = edge_index[0]
    dst = edge_index[1]

    e0p = jnp.pad(embedding, ((0, RT - N), (0, HP - H)))
    w0p = _pad_w(W_gc_0)
    b0p = _pad_b(b_gc_0)
    w1p = _pad_w(W_gc_1)
    b1p = _pad_b(b_gc_1)
    wc0p = _pad_w(W_concat[:H])
    wc1p = _pad_w(W_concat[H:2 * H])
    wc2p = _pad_w(W_concat[2 * H:])
    bcp = _pad_b(b_concat)
    w2p = _pad_w(nasr_w2)
    w3p = _pad_w(nasr_w3)
    vpp = _pad_b(nasr_v[0])
    nbp = _pad_b(nasr_b)

    agg1p = _scatter_layer(e0p, src, dst)
    e1p = _mm_act(agg1p, w0p, b0p)
    agg2p = _scatter_layer(e1p, src, dst)
    finalp, ssq_e0 = _final_stage(agg2p, e0p, e1p, w1p, b1p,
                                  wc0p, wc1p, wc2p, bcp)

    gidx = jnp.take_along_axis(item, alias, axis=1)
    seq_hp = _session_gather(finalp, gidx.reshape(-1))
    last_h = seq_hp.reshape(B, T, HP)[:, T - 1]
    valid = (item != 0).astype(jnp.float32)
    maskf = mask.reshape(B * T, 1)
    etp = finalp[:T]

    map_ = _session_stage(seq_hp, last_h, valid, maskf, etp,
                          w2p, w3p, vpp, nbp)

    labels = jnp.clip(tar - 1, 0, N - 2).astype(jnp.int32).reshape(B, 1)
    btabp = e0p[1:N]
    logits, rmax, rsum, rlab = _logits_stage(map_, btabp, labels)

    lse = jnp.log(rsum[:, 0]) + rmax[:, 0]
    ce = jnp.mean(lse - rlab[:, 0])
    wsum = (jnp.sum(W_gc_0 ** 2) + jnp.sum(b_gc_0 ** 2)
            + jnp.sum(W_gc_1 ** 2) + jnp.sum(b_gc_1 ** 2)
            + jnp.sum(W_concat ** 2) + jnp.sum(b_concat ** 2)
            + jnp.sum(nasr_w2 ** 2) + jnp.sum(nasr_w3 ** 2)
            + jnp.sum(nasr_v ** 2) + jnp.sum(nasr_b ** 2))
    l2 = 0.5 * (ssq_e0[0, 0] + wsum) * L2C
    loss = ce + l2
    return (loss, logits)


# speedup vs baseline: 2.3167x; 2.1853x over previous
import functools

import jax
import jax.numpy as jnp
from jax import lax
from jax.experimental import pallas as pl
from jax.experimental.pallas import tpu as pltpu
from jax.experimental.pallas import tpu_sc as plsc

N = 50000
H = 100
HP = 128
B = 256
T = 50
E = 800000
V = N - 1
L2C = 1e-05

RT = 51200
RB = 3200
CB = 2048


def _leaky(x):
    return jnp.where(x >= 0, x, 0.2 * x)


def _mm_act_body(x_ref, w_ref, b_ref, o_ref):
    o_ref[...] = _leaky(
        jnp.dot(x_ref[...], w_ref[...], preferred_element_type=jnp.float32, precision=lax.Precision.HIGHEST)
        + b_ref[...])


def _mm_act(x, w, b):
    return pl.pallas_call(
        _mm_act_body,
        grid=(RT // RB,),
        in_specs=[
            pl.BlockSpec((RB, HP), lambda i: (i, 0)),
            pl.BlockSpec((HP, HP), lambda i: (0, 0)),
            pl.BlockSpec((1, HP), lambda i: (0, 0)),
        ],
        out_specs=pl.BlockSpec((RB, HP), lambda i: (i, 0)),
        out_shape=jax.ShapeDtypeStruct((RT, HP), jnp.float32),
    )(x, w, b)


def _final_body(agg2_ref, e0_ref, e1_ref, w1_ref, b1_ref,
                wc0_ref, wc1_ref, wc2_ref, bc_ref, o_ref, ssq_ref):
    i = pl.program_id(0)
    e0 = e0_ref[...]
    e2 = _leaky(
        jnp.dot(agg2_ref[...], w1_ref[...], preferred_element_type=jnp.float32, precision=lax.Precision.HIGHEST)
        + b1_ref[...])
    f = (jnp.dot(e0, wc0_ref[...], preferred_element_type=jnp.float32, precision=lax.Precision.HIGHEST)
         + jnp.dot(e1_ref[...], wc1_ref[...], preferred_element_type=jnp.float32, precision=lax.Precision.HIGHEST)
         + jnp.dot(e2, wc2_ref[...], preferred_element_type=jnp.float32, precision=lax.Precision.HIGHEST)
         + bc_ref[...])
    o_ref[...] = _leaky(f)
    part = jnp.sum(e0 * e0).reshape(1, 1)

    @pl.when(i == 0)
    def _():
        ssq_ref[...] = part

    @pl.when(i > 0)
    def _():
        ssq_ref[...] += part


def _final_stage(agg2p, e0p, e1p, w1, b1, wc0, wc1, wc2, bc):
    return pl.pallas_call(
        _final_body,
        grid=(RT // RB,),
        in_specs=[
            pl.BlockSpec((RB, HP), lambda i: (i, 0)),
            pl.BlockSpec((RB, HP), lambda i: (i, 0)),
            pl.BlockSpec((RB, HP), lambda i: (i, 0)),
        ] + [pl.BlockSpec((HP, HP), lambda i: (0, 0))] * 1
          + [pl.BlockSpec((1, HP), lambda i: (0, 0))]
          + [pl.BlockSpec((HP, HP), lambda i: (0, 0))] * 3
          + [pl.BlockSpec((1, HP), lambda i: (0, 0))],
        out_specs=[
            pl.BlockSpec((RB, HP), lambda i: (i, 0)),
            pl.BlockSpec((1, 1), lambda i: (0, 0)),
        ],
        out_shape=[
            jax.ShapeDtypeStruct((RT, HP), jnp.float32),
            jax.ShapeDtypeStruct((1, 1), jnp.float32),
        ],
    )(agg2p, e0p, e1p, w1, b1, wc0, wc1, wc2, bc)


_SB = 32
_XB = _SB * T


def _session_body(x_ref, last_ref, valid_ref, maskf_ref, et_ref,
                  w2_ref, w3_ref, v_ref, nb_ref, ma_ref):
    x = x_ref[...]
    lasth = last_ref[...]
    ri = lax.broadcasted_iota(jnp.int32, (_XB, _SB), 0) // T
    ci = lax.broadcasted_iota(jnp.int32, (_XB, _SB), 1)
    sel = (ri == ci).astype(jnp.float32)
    lastb = jnp.dot(sel, lasth, preferred_element_type=jnp.float32, precision=lax.Precision.HIGHEST)
    seq = jnp.dot(x, w2_ref[...], preferred_element_type=jnp.float32, precision=lax.Precision.HIGHEST)
    m = jax.nn.sigmoid(lastb + seq + nb_ref[...])
    coef = jnp.dot(m, v_ref[...].T, preferred_element_type=jnp.float32, precision=lax.Precision.HIGHEST)
    coef = coef * maskf_ref[...]
    glb = lax.dot_general(sel, coef * x, (((0,), (0,)), ((), ())),
                          preferred_element_type=jnp.float32, precision=lax.Precision.HIGHEST)
    vald = valid_ref[...]
    cnt = jnp.clip(jnp.sum(vald, axis=1, keepdims=True), 1.0, None)
    last_emb = jnp.dot(vald, et_ref[...], preferred_element_type=jnp.float32, precision=lax.Precision.HIGHEST) / cnt
    gate = glb + jnp.dot(last_emb, w3_ref[...], preferred_element_type=jnp.float32, precision=lax.Precision.HIGHEST)
    ma_ref[...] = gate * lasth


def _session_stage(seq_hp, last_h, valid, maskf, etp, w2, w3, vp, nbp):
    return pl.pallas_call(
        _session_body,
        grid=(B // _SB,),
        in_specs=[
            pl.BlockSpec((_XB, HP), lambda i: (i, 0)),
            pl.BlockSpec((_SB, HP), lambda i: (i, 0)),
            pl.BlockSpec((_SB, T), lambda i: (i, 0)),
            pl.BlockSpec((_XB, 1), lambda i: (i, 0)),
            pl.BlockSpec((T, HP), lambda i: (0, 0)),
            pl.BlockSpec((HP, HP), lambda i: (0, 0)),
            pl.BlockSpec((HP, HP), lambda i: (0, 0)),
            pl.BlockSpec((1, HP), lambda i: (0, 0)),
            pl.BlockSpec((1, HP), lambda i: (0, 0)),
        ],
        out_specs=pl.BlockSpec((_SB, HP), lambda i: (i, 0)),
        out_shape=jax.ShapeDtypeStruct((B, HP), jnp.float32),
    )(seq_hp, last_h, valid, maskf, etp, w2, w3, vp, nbp)


def _logits_body(ma_ref, bt_ref, lab_ref, o_ref, mx_ref, sm_ref, lg_ref):
    i = pl.program_id(0)
    bt = bt_ref[...]
    lg = lax.dot_general(ma_ref[...], bt, (((1,), (1,)), ((), ())),
                         preferred_element_type=jnp.float32, precision=lax.Precision.HIGHEST)
    col = i * CB + lax.broadcasted_iota(jnp.int32, (B, CB), 1)
    okc = col < V
    lgm = jnp.where(okc, lg, -1e30)
    o_ref[...] = lg
    bm = jnp.max(lgm, axis=1, keepdims=True)
    bs = jnp.sum(jnp.exp(lgm - bm), axis=1, keepdims=True)
    labs = lab_ref[...]
    hit = jnp.where((col == labs) & okc, lg, 0.0)
    lab_part = jnp.sum(hit, axis=1, keepdims=True)

    @pl.when(i == 0)
    def _():
        mx_ref[...] = bm
        sm_ref[...] = bs
        lg_ref[...] = lab_part

    @pl.when(i > 0)
    def _():
        m_old = mx_ref[...]
        m_new = jnp.maximum(m_old, bm)
        sm_ref[...] = (sm_ref[...] * jnp.exp(m_old - m_new)
                       + bs * jnp.exp(bm - m_new))
        mx_ref[...] = m_new
        lg_ref[...] += lab_part


def _logits_stage(map_, btabp, labels):
    nblk = pl.cdiv(V, CB)
    return pl.pallas_call(
        _logits_body,
        grid=(nblk,),
        in_specs=[
            pl.BlockSpec((B, HP), lambda i: (0, 0)),
            pl.BlockSpec((CB, HP), lambda i: (i, 0)),
            pl.BlockSpec((B, 1), lambda i: (0, 0)),
        ],
        out_specs=[
            pl.BlockSpec((B, CB), lambda i: (0, i)),
            pl.BlockSpec((B, 1), lambda i: (0, 0)),
            pl.BlockSpec((B, 1), lambda i: (0, 0)),
            pl.BlockSpec((B, 1), lambda i: (0, 0)),
        ],
        out_shape=[
            jax.ShapeDtypeStruct((B, V), jnp.float32),
            jax.ShapeDtypeStruct((B, 1), jnp.float32),
            jax.ShapeDtypeStruct((B, 1), jnp.float32),
            jax.ShapeDtypeStruct((B, 1), jnp.float32),
        ],
    )(map_, btabp, labels)


_NC, _NS = 2, 16
_CH = 128
_EPAD = 802816
_ET = _EPAD // _NS
_NCHUNK = _ET // _CH
_TRW = RT // _NS
_DR = 128


def _sc_mesh():
    return plsc.VectorSubcoreMesh(core_axis_name="c", subcore_axis_name="s",
                                  num_cores=_NC, num_subcores=_NS)


def _sc_scatter_body(ego_ref, src_ref, dst_ref, zr_ref, out_ref,
                     acc, si, di, gi, rows, dbuf, didx, zbuf, sem):
    c = lax.axis_index("c")
    s = lax.axis_index("s")
    iot = lax.iota(jnp.int32, 16)
    pltpu.sync_copy(zr_ref, zbuf)
    for p in range(2):
        chunk = 2 * p + c

        def zacc(k, _):
            pltpu.sync_copy(zbuf, acc.at[pl.ds(s * _TRW + k * _DR, _DR)])
            return 0
        lax.fori_loop(0, _TRW // _DR, zacc, 0)
        plsc.subcore_barrier()

        def ebody(j, _):
            eb = s * _ET + j * _CH
            pltpu.sync_copy(src_ref.at[pl.ds(eb, _CH)], si)
            pltpu.sync_copy(dst_ref.at[pl.ds(eb, _CH)], di)
            for g in range(_CH // 16):
                gi[pl.ds(g * 16, 16)] = si[pl.ds(g * 16, 16)] * 4 + chunk
            pltpu.async_copy(ego_ref.at[gi], rows, sem).wait()
            pltpu.sync_copy(rows, acc.at[di], add=True)
            return 0
        lax.fori_loop(0, _NCHUNK, ebody, 0)
        plsc.subcore_barrier()

        def dbody(k, _):
            rb = s * _TRW + k * _DR
            pltpu.sync_copy(acc.at[pl.ds(rb, _DR)], dbuf)
            for g in range(_DR // 16):
                didx[pl.ds(g * 16, 16)] = 4 * rb + 64 * g + 4 * iot + chunk
            pltpu.async_copy(dbuf, out_ref.at[didx], sem).wait()
            return 0
        lax.fori_loop(0, _TRW // _DR, dbody, 0)
        plsc.subcore_barrier()


def _scatter_layer(ep, srcp, dstp, zr):
    ego_v = ep.reshape(RT * 4, 32)
    run = functools.partial(
        pl.kernel,
        out_type=jax.ShapeDtypeStruct((RT * 4, 32), jnp.float32),
        mesh=_sc_mesh(),
        compiler_params=pltpu.CompilerParams(use_tc_tiling_on_sc=False),
        scratch_types=[
            pltpu.VMEM_SHARED((RT, 32), jnp.float32),
            pltpu.VMEM((_CH,), jnp.int32),
            pltpu.VMEM((_CH,), jnp.int32),
            pltpu.VMEM((_CH,), jnp.int32),
            pltpu.VMEM((_CH, 32), jnp.float32),
            pltpu.VMEM((_DR, 32), jnp.float32),
            pltpu.VMEM((_DR,), jnp.int32),
            pltpu.VMEM((_DR, 32), jnp.float32),
            pltpu.SemaphoreType.DMA,
        ],
    )(_sc_scatter_body)
    return run(ego_v, srcp, dstp, zr).reshape(RT, HP)


_GW = (B * T) // (_NC * _NS)


def _sc_gather_body(tab_ref, gi_ref, out_ref, i128, i16, r128, r16, sem):
    cc = lax.axis_index("c")
    ss = lax.axis_index("s")
    base = (ss * _NC + cc) * _GW
    for off in (0, 128, 256):
        pltpu.sync_copy(gi_ref.at[pl.ds(base + off, 128)], i128)
        pltpu.async_copy(tab_ref.at[i128], r128, sem).wait()
        pltpu.sync_copy(r128, out_ref.at[pl.ds(base + off, 128)])
    pltpu.sync_copy(gi_ref.at[pl.ds(base + 384, 16)], i16)
    pltpu.async_copy(tab_ref.at[i16], r16, sem).wait()
    pltpu.sync_copy(r16, out_ref.at[pl.ds(base + 384, 16)])


def _session_gather(finalp, gidx):
    run = functools.partial(
        pl.kernel,
        out_type=jax.ShapeDtypeStruct((B * T, HP), jnp.float32),
        mesh=_sc_mesh(),
        scratch_types=[
            pltpu.VMEM((128,), jnp.int32),
            pltpu.VMEM((16,), jnp.int32),
            pltpu.VMEM((128, HP), jnp.float32),
            pltpu.VMEM((16, HP), jnp.float32),
            pltpu.SemaphoreType.DMA,
        ],
    )(_sc_gather_body)
    return run(finalp, gidx)


def _pad_w(w):
    return jnp.pad(w, ((0, HP - w.shape[0]), (0, HP - w.shape[1])))


def _pad_b(b):
    return jnp.pad(b.reshape(1, -1), ((0, 0), (0, HP - b.shape[-1])))


@jax.jit
def kernel(edge_index, item, alias, mask, tar, embedding,
           W_gc_0, b_gc_0, W_gc_1, b_gc_1, W_concat, b_concat,
           nasr_w2, nasr_w3, nasr_v, nasr_b):
    npad = _EPAD - E
    srcp = jnp.concatenate([edge_index[0],
                            (jnp.arange(npad, dtype=jnp.int32) * 131) % N])
    dstp = jnp.concatenate([edge_index[1],
                            N + (jnp.arange(npad, dtype=jnp.int32) % (RT - N))])
    zr = jnp.zeros((_DR, 32), jnp.float32)

    e0p = jnp.pad(embedding, ((0, RT - N), (0, HP - H)))
    w0p = _pad_w(W_gc_0)
    b0p = _pad_b(b_gc_0)
    w1p = _pad_w(W_gc_1)
    b1p = _pad_b(b_gc_1)
    wc0p = _pad_w(W_concat[:H])
    wc1p = _pad_w(W_concat[H:2 * H])
    wc2p = _pad_w(W_concat[2 * H:])
    bcp = _pad_b(b_concat)
    w2p = _pad_w(nasr_w2)
    w3p = _pad_w(nasr_w3)
    vpp = _pad_b(nasr_v[0])
    nbp = _pad_b(nasr_b)

    agg1p = _scatter_layer(e0p, srcp, dstp, zr)
    e1p = _mm_act(agg1p, w0p, b0p)
    agg2p = _scatter_layer(e1p, srcp, dstp, zr)
    finalp, ssq_e0 = _final_stage(agg2p, e0p, e1p, w1p, b1p,
                                  wc0p, wc1p, wc2p, bcp)

    gidx = jnp.take_along_axis(item, alias, axis=1)
    seq_hp = _session_gather(finalp, gidx.reshape(-1))
    last_h = seq_hp.reshape(B, T, HP)[:, T - 1]
    valid = (item != 0).astype(jnp.float32)
    maskf = mask.reshape(B * T, 1)
    etp = finalp[:T]

    map_ = _session_stage(seq_hp, last_h, valid, maskf, etp,
                          w2p, w3p, vpp, nbp)

    labels = jnp.clip(tar - 1, 0, N - 2).astype(jnp.int32).reshape(B, 1)
    btabp = e0p[1:N]
    logits, rmax, rsum, rlab = _logits_stage(map_, btabp, labels)

    lse = jnp.log(rsum[:, 0]) + rmax[:, 0]
    ce = jnp.mean(lse - rlab[:, 0])
    wsum = (jnp.sum(W_gc_0 ** 2) + jnp.sum(b_gc_0 ** 2)
            + jnp.sum(W_gc_1 ** 2) + jnp.sum(b_gc_1 ** 2)
            + jnp.sum(W_concat ** 2) + jnp.sum(b_concat ** 2)
            + jnp.sum(nasr_w2 ** 2) + jnp.sum(nasr_w3 ** 2)
            + jnp.sum(nasr_v ** 2) + jnp.sum(nasr_b ** 2))
    l2 = 0.5 * (ssq_e0[0, 0] + wsum) * L2C
    loss = ce + l2
    return (loss, logits)


# speedup vs baseline: 6.4834x; 2.7985x over previous
import functools

import jax
import jax.numpy as jnp
from jax import lax
from jax.experimental import pallas as pl
from jax.experimental.pallas import tpu as pltpu
from jax.experimental.pallas import tpu_sc as plsc

N = 50000
H = 100
HP = 128
B = 256
T = 50
E = 800000
V = N - 1
L2C = 1e-05

RT = 50176
RB = 3136
CB = 2048


def _leaky(x):
    return jnp.where(x >= 0, x, 0.2 * x)


def _mm_act_body(x_ref, w_ref, b_ref, o_ref):
    o_ref[...] = _leaky(
        jnp.dot(x_ref[...], w_ref[...], preferred_element_type=jnp.float32, precision=lax.Precision.HIGHEST)
        + b_ref[...])


def _mm_act(x, w, b):
    return pl.pallas_call(
        _mm_act_body,
        grid=(RT // RB,),
        in_specs=[
            pl.BlockSpec((RB, HP), lambda i: (i, 0)),
            pl.BlockSpec((HP, HP), lambda i: (0, 0)),
            pl.BlockSpec((1, HP), lambda i: (0, 0)),
        ],
        out_specs=pl.BlockSpec((RB, HP), lambda i: (i, 0)),
        out_shape=jax.ShapeDtypeStruct((RT, HP), jnp.float32),
    )(x, w, b)


def _final_body(agg2_ref, e0_ref, e1_ref, w1_ref, b1_ref,
                wc0_ref, wc1_ref, wc2_ref, bc_ref, o_ref, ssq_ref):
    i = pl.program_id(0)
    e0 = e0_ref[...]
    e2 = _leaky(
        jnp.dot(agg2_ref[...], w1_ref[...], preferred_element_type=jnp.float32, precision=lax.Precision.HIGHEST)
        + b1_ref[...])
    f = (jnp.dot(e0, wc0_ref[...], preferred_element_type=jnp.float32, precision=lax.Precision.HIGHEST)
         + jnp.dot(e1_ref[...], wc1_ref[...], preferred_element_type=jnp.float32, precision=lax.Precision.HIGHEST)
         + jnp.dot(e2, wc2_ref[...], preferred_element_type=jnp.float32, precision=lax.Precision.HIGHEST)
         + bc_ref[...])
    o_ref[...] = _leaky(f)
    part = jnp.sum(e0 * e0).reshape(1, 1)

    @pl.when(i == 0)
    def _():
        ssq_ref[...] = part

    @pl.when(i > 0)
    def _():
        ssq_ref[...] += part


def _final_stage(agg2p, e0p, e1p, w1, b1, wc0, wc1, wc2, bc):
    return pl.pallas_call(
        _final_body,
        grid=(RT // RB,),
        in_specs=[
            pl.BlockSpec((RB, HP), lambda i: (i, 0)),
            pl.BlockSpec((RB, HP), lambda i: (i, 0)),
            pl.BlockSpec((RB, HP), lambda i: (i, 0)),
        ] + [pl.BlockSpec((HP, HP), lambda i: (0, 0))] * 1
          + [pl.BlockSpec((1, HP), lambda i: (0, 0))]
          + [pl.BlockSpec((HP, HP), lambda i: (0, 0))] * 3
          + [pl.BlockSpec((1, HP), lambda i: (0, 0))],
        out_specs=[
            pl.BlockSpec((RB, HP), lambda i: (i, 0)),
            pl.BlockSpec((1, 1), lambda i: (0, 0)),
        ],
        out_shape=[
            jax.ShapeDtypeStruct((RT, HP), jnp.float32),
            jax.ShapeDtypeStruct((1, 1), jnp.float32),
        ],
    )(agg2p, e0p, e1p, w1, b1, wc0, wc1, wc2, bc)


_SB = 32
_XB = _SB * T


def _session_body(x_ref, last_ref, valid_ref, maskf_ref, et_ref,
                  w2_ref, w3_ref, v_ref, nb_ref, ma_ref):
    x = x_ref[...]
    lasth = last_ref[...]
    ri = lax.broadcasted_iota(jnp.int32, (_XB, _SB), 0) // T
    ci = lax.broadcasted_iota(jnp.int32, (_XB, _SB), 1)
    sel = (ri == ci).astype(jnp.float32)
    lastb = jnp.dot(sel, lasth, preferred_element_type=jnp.float32, precision=lax.Precision.HIGHEST)
    seq = jnp.dot(x, w2_ref[...], preferred_element_type=jnp.float32, precision=lax.Precision.HIGHEST)
    m = jax.nn.sigmoid(lastb + seq + nb_ref[...])
    coef = jnp.dot(m, v_ref[...].T, preferred_element_type=jnp.float32, precision=lax.Precision.HIGHEST)
    coef = coef * maskf_ref[...]
    glb = lax.dot_general(sel, coef * x, (((0,), (0,)), ((), ())),
                          preferred_element_type=jnp.float32, precision=lax.Precision.HIGHEST)
    vald = valid_ref[...]
    cnt = jnp.clip(jnp.sum(vald, axis=1, keepdims=True), 1.0, None)
    last_emb = jnp.dot(vald, et_ref[...], preferred_element_type=jnp.float32, precision=lax.Precision.HIGHEST) / cnt
    gate = glb + jnp.dot(last_emb, w3_ref[...], preferred_element_type=jnp.float32, precision=lax.Precision.HIGHEST)
    ma_ref[...] = gate * lasth


def _session_stage(seq_hp, last_h, valid, maskf, etp, w2, w3, vp, nbp):
    return pl.pallas_call(
        _session_body,
        grid=(B // _SB,),
        in_specs=[
            pl.BlockSpec((_XB, HP), lambda i: (i, 0)),
            pl.BlockSpec((_SB, HP), lambda i: (i, 0)),
            pl.BlockSpec((_SB, T), lambda i: (i, 0)),
            pl.BlockSpec((_XB, 1), lambda i: (i, 0)),
            pl.BlockSpec((T, HP), lambda i: (0, 0)),
            pl.BlockSpec((HP, HP), lambda i: (0, 0)),
            pl.BlockSpec((HP, HP), lambda i: (0, 0)),
            pl.BlockSpec((1, HP), lambda i: (0, 0)),
            pl.BlockSpec((1, HP), lambda i: (0, 0)),
        ],
        out_specs=pl.BlockSpec((_SB, HP), lambda i: (i, 0)),
        out_shape=jax.ShapeDtypeStruct((B, HP), jnp.float32),
    )(seq_hp, last_h, valid, maskf, etp, w2, w3, vp, nbp)


def _logits_body(ma_ref, bt_ref, lab_ref, o_ref, mx_ref, sm_ref, lg_ref):
    i = pl.program_id(0)
    bt = bt_ref[...]
    lg = lax.dot_general(ma_ref[...], bt, (((1,), (1,)), ((), ())),
                         preferred_element_type=jnp.float32, precision=lax.Precision.HIGHEST)
    col = i * CB + lax.broadcasted_iota(jnp.int32, (B, CB), 1)
    okc = col < V
    lgm = jnp.where(okc, lg, -1e30)
    o_ref[...] = lg
    bm = jnp.max(lgm, axis=1, keepdims=True)
    bs = jnp.sum(jnp.exp(lgm - bm), axis=1, keepdims=True)
    labs = lab_ref[...]
    hit = jnp.where((col == labs) & okc, lg, 0.0)
    lab_part = jnp.sum(hit, axis=1, keepdims=True)

    @pl.when(i == 0)
    def _():
        mx_ref[...] = bm
        sm_ref[...] = bs
        lg_ref[...] = lab_part

    @pl.when(i > 0)
    def _():
        m_old = mx_ref[...]
        m_new = jnp.maximum(m_old, bm)
        sm_ref[...] = (sm_ref[...] * jnp.exp(m_old - m_new)
                       + bs * jnp.exp(bm - m_new))
        mx_ref[...] = m_new
        lg_ref[...] += lab_part


def _logits_stage(map_, btabp, labels):
    nblk = pl.cdiv(V, CB)
    return pl.pallas_call(
        _logits_body,
        grid=(nblk,),
        in_specs=[
            pl.BlockSpec((B, HP), lambda i: (0, 0)),
            pl.BlockSpec((CB, HP), lambda i: (i, 0)),
            pl.BlockSpec((B, 1), lambda i: (0, 0)),
        ],
        out_specs=[
            pl.BlockSpec((B, CB), lambda i: (0, i)),
            pl.BlockSpec((B, 1), lambda i: (0, 0)),
            pl.BlockSpec((B, 1), lambda i: (0, 0)),
            pl.BlockSpec((B, 1), lambda i: (0, 0)),
        ],
        out_shape=[
            jax.ShapeDtypeStruct((B, V), jnp.float32),
            jax.ShapeDtypeStruct((B, 1), jnp.float32),
            jax.ShapeDtypeStruct((B, 1), jnp.float32),
            jax.ShapeDtypeStruct((B, 1), jnp.float32),
        ],
    )(map_, btabp, labels)


_NC, _NS = 2, 16
_CH = 128
_EPAD = 802816
_ET = _EPAD // _NS
_NCHUNK = _ET // _CH
_TRW = RT // _NS
_DR = 112


def _sc_mesh():
    return plsc.VectorSubcoreMesh(core_axis_name="c", subcore_axis_name="s",
                                  num_cores=_NC, num_subcores=_NS)


_NB = 4
_NG = _NCHUNK // _NB


def _sc_scatter_body(ego_ref, src_ref, dst_ref, zr_ref, out_ref,
                     acc, si, di, gi, rows, dbuf, didx, zbuf,
                     semi, semg, sems, semd):
    c = lax.axis_index("c")
    s = lax.axis_index("s")
    iot = lax.iota(jnp.int32, 16)
    pltpu.sync_copy(zr_ref, zbuf)

    def fire_idx(jc, b):
        eb = s * _ET + jc * _CH
        pltpu.async_copy(src_ref.at[pl.ds(eb, _CH)], si.at[b], semi.at[b])
        pltpu.async_copy(dst_ref.at[pl.ds(eb, _CH)], di.at[b], semi.at[b])

    for p in range(2):
        chunk = 2 * p + c

        if p == 0:
            def zacc(k, _):
                pltpu.sync_copy(zbuf, acc.at[pl.ds(s * _TRW + k * _DR, _DR)])
                return 0
            lax.fori_loop(0, _TRW // _DR, zacc, 0)
        plsc.subcore_barrier()

        for b in range(_NB):
            fire_idx(b, b)

        def group(g, _):
            gds = []
            for b in range(_NB):
                pltpu.make_async_copy(src_ref.at[pl.ds(0, _CH)],
                                      si.at[b], semi.at[b]).wait()
                pltpu.make_async_copy(dst_ref.at[pl.ds(0, _CH)],
                                      di.at[b], semi.at[b]).wait()
                for gg in range(_CH // 16):
                    sl = pl.ds(gg * 16, 16)
                    gi[b, sl] = si[b, sl] * 4 + chunk
                gds.append(pltpu.async_copy(ego_ref.at[gi.at[b]],
                                            rows.at[b], semg.at[b]))
            sds = []
            for b in range(_NB):
                gds[b].wait()
                sds.append(pltpu.async_copy(rows.at[b], acc.at[di.at[b]],
                                            sems.at[b], add=True))
            for b in range(_NB):
                sds[b].wait()

                @pl.when(g < _NG - 1)
                def _():
                    fire_idx((g + 1) * _NB + b, b)
            return 0
        lax.fori_loop(0, _NG, group, 0)
        plsc.subcore_barrier()

        def dbody(k, _):
            rb = s * _TRW + k * _DR
            pltpu.sync_copy(acc.at[pl.ds(rb, _DR)], dbuf)
            if p == 0:
                pltpu.sync_copy(zbuf, acc.at[pl.ds(rb, _DR)])
            for gg in range(_DR // 16):
                didx[pl.ds(gg * 16, 16)] = 4 * rb + 64 * gg + 4 * iot + chunk
            pltpu.async_copy(dbuf, out_ref.at[didx], semd).wait()
            return 0
        lax.fori_loop(0, _TRW // _DR, dbody, 0)


def _scatter_layer(ep, srcp, dstp, zr):
    ego_v = ep.reshape(RT * 4, 32)
    run = functools.partial(
        pl.kernel,
        out_type=jax.ShapeDtypeStruct((RT * 4, 32), jnp.float32),
        mesh=_sc_mesh(),
        compiler_params=pltpu.CompilerParams(use_tc_tiling_on_sc=False),
        scratch_types=[
            pltpu.VMEM_SHARED((RT, 32), jnp.float32),
            pltpu.VMEM((_NB, _CH), jnp.int32),
            pltpu.VMEM((_NB, _CH), jnp.int32),
            pltpu.VMEM((_NB, _CH), jnp.int32),
            pltpu.VMEM((_NB, _CH, 32), jnp.float32),
            pltpu.VMEM((_DR, 32), jnp.float32),
            pltpu.VMEM((_DR,), jnp.int32),
            pltpu.VMEM((_DR, 32), jnp.float32),
            pltpu.SemaphoreType.DMA((_NB,)),
            pltpu.SemaphoreType.DMA((_NB,)),
            pltpu.SemaphoreType.DMA((_NB,)),
            pltpu.SemaphoreType.DMA,
        ],
    )(_sc_scatter_body)
    return run(ego_v, srcp, dstp, zr).reshape(RT, HP)


_GW = (B * T) // (_NC * _NS)


def _sc_gather_body(tab_ref, gi_ref, out_ref, i128, i16, r128, r16, sem):
    cc = lax.axis_index("c")
    ss = lax.axis_index("s")
    base = (ss * _NC + cc) * _GW
    for off in (0, 128, 256):
        pltpu.sync_copy(gi_ref.at[pl.ds(base + off, 128)], i128)
        pltpu.async_copy(tab_ref.at[i128], r128, sem).wait()
        pltpu.sync_copy(r128, out_ref.at[pl.ds(base + off, 128)])
    pltpu.sync_copy(gi_ref.at[pl.ds(base + 384, 16)], i16)
    pltpu.async_copy(tab_ref.at[i16], r16, sem).wait()
    pltpu.sync_copy(r16, out_ref.at[pl.ds(base + 384, 16)])


def _session_gather(finalp, gidx):
    run = functools.partial(
        pl.kernel,
        out_type=jax.ShapeDtypeStruct((B * T, HP), jnp.float32),
        mesh=_sc_mesh(),
        scratch_types=[
            pltpu.VMEM((128,), jnp.int32),
            pltpu.VMEM((16,), jnp.int32),
            pltpu.VMEM((128, HP), jnp.float32),
            pltpu.VMEM((16, HP), jnp.float32),
            pltpu.SemaphoreType.DMA,
        ],
    )(_sc_gather_body)
    return run(finalp, gidx)


def _pad_w(w):
    return jnp.pad(w, ((0, HP - w.shape[0]), (0, HP - w.shape[1])))


def _pad_b(b):
    return jnp.pad(b.reshape(1, -1), ((0, 0), (0, HP - b.shape[-1])))


@jax.jit
def kernel(edge_index, item, alias, mask, tar, embedding,
           W_gc_0, b_gc_0, W_gc_1, b_gc_1, W_concat, b_concat,
           nasr_w2, nasr_w3, nasr_v, nasr_b):
    npad = _EPAD - E
    srcp = jnp.concatenate([edge_index[0],
                            (jnp.arange(npad, dtype=jnp.int32) * 131) % N])
    dstp = jnp.concatenate([edge_index[1],
                            N + (jnp.arange(npad, dtype=jnp.int32) % (RT - N))])
    zr = jnp.zeros((_DR, 32), jnp.float32)

    e0p = jnp.pad(embedding, ((0, RT - N), (0, HP - H)))
    w0p = _pad_w(W_gc_0)
    b0p = _pad_b(b_gc_0)
    w1p = _pad_w(W_gc_1)
    b1p = _pad_b(b_gc_1)
    wc0p = _pad_w(W_concat[:H])
    wc1p = _pad_w(W_concat[H:2 * H])
    wc2p = _pad_w(W_concat[2 * H:])
    bcp = _pad_b(b_concat)
    w2p = _pad_w(nasr_w2)
    w3p = _pad_w(nasr_w3)
    vpp = _pad_b(nasr_v[0])
    nbp = _pad_b(nasr_b)

    agg1p = _scatter_layer(e0p, srcp, dstp, zr)
    e1p = _mm_act(agg1p, w0p, b0p)
    agg2p = _scatter_layer(e1p, srcp, dstp, zr)
    finalp, ssq_e0 = _final_stage(agg2p, e0p, e1p, w1p, b1p,
                                  wc0p, wc1p, wc2p, bcp)

    gidx = jnp.take_along_axis(item, alias, axis=1)
    seq_hp = _session_gather(finalp, gidx.reshape(-1))
    last_h = seq_hp.reshape(B, T, HP)[:, T - 1]
    valid = (item != 0).astype(jnp.float32)
    maskf = mask.reshape(B * T, 1)
    etp = finalp[:T]

    map_ = _session_stage(seq_hp, last_h, valid, maskf, etp,
                          w2p, w3p, vpp, nbp)

    labels = jnp.clip(tar - 1, 0, N - 2).astype(jnp.int32).reshape(B, 1)
    btabp = e0p[1:N]
    logits, rmax, rsum, rlab = _logits_stage(map_, btabp, labels)

    lse = jnp.log(rsum[:, 0]) + rmax[:, 0]
    ce = jnp.mean(lse - rlab[:, 0])
    wsum = (jnp.sum(W_gc_0 ** 2) + jnp.sum(b_gc_0 ** 2)
            + jnp.sum(W_gc_1 ** 2) + jnp.sum(b_gc_1 ** 2)
            + jnp.sum(W_concat ** 2) + jnp.sum(b_concat ** 2)
            + jnp.sum(nasr_w2 ** 2) + jnp.sum(nasr_w3 ** 2)
            + jnp.sum(nasr_v ** 2) + jnp.sum(nasr_b ** 2))
    l2 = 0.5 * (ssq_e0[0, 0] + wsum) * L2C
    loss = ce + l2
    return (loss, logits)


# speedup vs baseline: 7.0599x; 1.0889x over previous
import functools

import jax
import jax.numpy as jnp
from jax import lax
from jax.experimental import pallas as pl
from jax.experimental.pallas import tpu as pltpu
from jax.experimental.pallas import tpu_sc as plsc

N = 50000
H = 100
HP = 128
B = 256
T = 50
E = 800000
V = N - 1
L2C = 1e-05

RT = 50176
RB = 3136
CB = 2048


def _leaky(x):
    return jnp.where(x >= 0, x, 0.2 * x)


def _mm_act_body(x_ref, w_ref, b_ref, o_ref):
    o_ref[...] = _leaky(
        jnp.dot(x_ref[...], w_ref[...], preferred_element_type=jnp.float32, precision=lax.Precision.HIGHEST)
        + b_ref[...])


def _mm_act(x, w, b):
    return pl.pallas_call(
        _mm_act_body,
        grid=(RT // RB,),
        in_specs=[
            pl.BlockSpec((RB, HP), lambda i: (i, 0)),
            pl.BlockSpec((HP, HP), lambda i: (0, 0)),
            pl.BlockSpec((1, HP), lambda i: (0, 0)),
        ],
        out_specs=pl.BlockSpec((RB, HP), lambda i: (i, 0)),
        out_shape=jax.ShapeDtypeStruct((RT, HP), jnp.float32),
    )(x, w, b)


def _final_body(agg2_ref, e0_ref, e1_ref, w1_ref, b1_ref,
                wc0_ref, wc1_ref, wc2_ref, bc_ref, o_ref, ssq_ref):
    i = pl.program_id(0)
    e0 = e0_ref[...]
    e2 = _leaky(
        jnp.dot(agg2_ref[...], w1_ref[...], preferred_element_type=jnp.float32, precision=lax.Precision.HIGHEST)
        + b1_ref[...])
    f = (jnp.dot(e0, wc0_ref[...], preferred_element_type=jnp.float32, precision=lax.Precision.HIGHEST)
         + jnp.dot(e1_ref[...], wc1_ref[...], preferred_element_type=jnp.float32, precision=lax.Precision.HIGHEST)
         + jnp.dot(e2, wc2_ref[...], preferred_element_type=jnp.float32, precision=lax.Precision.HIGHEST)
         + bc_ref[...])
    o_ref[...] = _leaky(f)
    part = jnp.sum(e0 * e0).reshape(1, 1)

    @pl.when(i == 0)
    def _():
        ssq_ref[...] = part

    @pl.when(i > 0)
    def _():
        ssq_ref[...] += part


def _final_stage(agg2p, e0p, e1p, w1, b1, wc0, wc1, wc2, bc):
    return pl.pallas_call(
        _final_body,
        grid=(RT // RB,),
        in_specs=[
            pl.BlockSpec((RB, HP), lambda i: (i, 0)),
            pl.BlockSpec((RB, HP), lambda i: (i, 0)),
            pl.BlockSpec((RB, HP), lambda i: (i, 0)),
        ] + [pl.BlockSpec((HP, HP), lambda i: (0, 0))] * 1
          + [pl.BlockSpec((1, HP), lambda i: (0, 0))]
          + [pl.BlockSpec((HP, HP), lambda i: (0, 0))] * 3
          + [pl.BlockSpec((1, HP), lambda i: (0, 0))],
        out_specs=[
            pl.BlockSpec((RB, HP), lambda i: (i, 0)),
            pl.BlockSpec((1, 1), lambda i: (0, 0)),
        ],
        out_shape=[
            jax.ShapeDtypeStruct((RT, HP), jnp.float32),
            jax.ShapeDtypeStruct((1, 1), jnp.float32),
        ],
    )(agg2p, e0p, e1p, w1, b1, wc0, wc1, wc2, bc)


_SB = 32
_XB = _SB * T


def _session_body(x_ref, last_ref, valid_ref, maskf_ref, et_ref,
                  w2_ref, w3_ref, v_ref, nb_ref, ma_ref):
    x = x_ref[...]
    lasth = last_ref[...]
    ri = lax.broadcasted_iota(jnp.int32, (_XB, _SB), 0) // T
    ci = lax.broadcasted_iota(jnp.int32, (_XB, _SB), 1)
    sel = (ri == ci).astype(jnp.float32)
    lastb = jnp.dot(sel, lasth, preferred_element_type=jnp.float32, precision=lax.Precision.HIGHEST)
    seq = jnp.dot(x, w2_ref[...], preferred_element_type=jnp.float32, precision=lax.Precision.HIGHEST)
    m = jax.nn.sigmoid(lastb + seq + nb_ref[...])
    coef = jnp.dot(m, v_ref[...].T, preferred_element_type=jnp.float32, precision=lax.Precision.HIGHEST)
    coef = coef * maskf_ref[...]
    glb = lax.dot_general(sel, coef * x, (((0,), (0,)), ((), ())),
                          preferred_element_type=jnp.float32, precision=lax.Precision.HIGHEST)
    vald = valid_ref[...]
    cnt = jnp.clip(jnp.sum(vald, axis=1, keepdims=True), 1.0, None)
    last_emb = jnp.dot(vald, et_ref[...], preferred_element_type=jnp.float32, precision=lax.Precision.HIGHEST) / cnt
    gate = glb + jnp.dot(last_emb, w3_ref[...], preferred_element_type=jnp.float32, precision=lax.Precision.HIGHEST)
    ma_ref[...] = gate * lasth


def _session_stage(seq_hp, last_h, valid, maskf, etp, w2, w3, vp, nbp):
    return pl.pallas_call(
        _session_body,
        grid=(B // _SB,),
        in_specs=[
            pl.BlockSpec((_XB, HP), lambda i: (i, 0)),
            pl.BlockSpec((_SB, HP), lambda i: (i, 0)),
            pl.BlockSpec((_SB, T), lambda i: (i, 0)),
            pl.BlockSpec((_XB, 1), lambda i: (i, 0)),
            pl.BlockSpec((T, HP), lambda i: (0, 0)),
            pl.BlockSpec((HP, HP), lambda i: (0, 0)),
            pl.BlockSpec((HP, HP), lambda i: (0, 0)),
            pl.BlockSpec((1, HP), lambda i: (0, 0)),
            pl.BlockSpec((1, HP), lambda i: (0, 0)),
        ],
        out_specs=pl.BlockSpec((_SB, HP), lambda i: (i, 0)),
        out_shape=jax.ShapeDtypeStruct((B, HP), jnp.float32),
    )(seq_hp, last_h, valid, maskf, etp, w2, w3, vp, nbp)


def _logits_body(ma_ref, bt_ref, lab_ref, o_ref, mx_ref, sm_ref, lg_ref):
    i = pl.program_id(0)
    bt = bt_ref[...]
    lg = lax.dot_general(ma_ref[...], bt, (((1,), (1,)), ((), ())),
                         preferred_element_type=jnp.float32, precision=lax.Precision.HIGHEST)
    col = i * CB + lax.broadcasted_iota(jnp.int32, (B, CB), 1)
    okc = col < V
    lgm = jnp.where(okc, lg, -1e30)
    o_ref[...] = lg
    bm = jnp.max(lgm, axis=1, keepdims=True)
    bs = jnp.sum(jnp.exp(lgm - bm), axis=1, keepdims=True)
    labs = lab_ref[...]
    hit = jnp.where((col == labs) & okc, lg, 0.0)
    lab_part = jnp.sum(hit, axis=1, keepdims=True)

    @pl.when(i == 0)
    def _():
        mx_ref[...] = bm
        sm_ref[...] = bs
        lg_ref[...] = lab_part

    @pl.when(i > 0)
    def _():
        m_old = mx_ref[...]
        m_new = jnp.maximum(m_old, bm)
        sm_ref[...] = (sm_ref[...] * jnp.exp(m_old - m_new)
                       + bs * jnp.exp(bm - m_new))
        mx_ref[...] = m_new
        lg_ref[...] += lab_part


def _logits_stage(map_, btabp, labels):
    nblk = pl.cdiv(V, CB)
    return pl.pallas_call(
        _logits_body,
        grid=(nblk,),
        in_specs=[
            pl.BlockSpec((B, HP), lambda i: (0, 0)),
            pl.BlockSpec((CB, HP), lambda i: (i, 0)),
            pl.BlockSpec((B, 1), lambda i: (0, 0)),
        ],
        out_specs=[
            pl.BlockSpec((B, CB), lambda i: (0, i)),
            pl.BlockSpec((B, 1), lambda i: (0, 0)),
            pl.BlockSpec((B, 1), lambda i: (0, 0)),
            pl.BlockSpec((B, 1), lambda i: (0, 0)),
        ],
        out_shape=[
            jax.ShapeDtypeStruct((B, V), jnp.float32),
            jax.ShapeDtypeStruct((B, 1), jnp.float32),
            jax.ShapeDtypeStruct((B, 1), jnp.float32),
            jax.ShapeDtypeStruct((B, 1), jnp.float32),
        ],
    )(map_, btabp, labels)


_NC, _NS = 2, 16
_CH = 128
_EPAD = 811008
_ET = _EPAD // _NS
_NCHUNK = _ET // _CH
_TRW = RT // _NS
_DR = 112


def _sc_mesh():
    return plsc.VectorSubcoreMesh(core_axis_name="c", subcore_axis_name="s",
                                  num_cores=_NC, num_subcores=_NS)


_NB = 6
_NG = _NCHUNK // _NB


def _sc_scatter_body(ego_ref, src_ref, dst_ref, out_ref,
                     acc, si, di, gi, rows, didx,
                     semi, semg, sems, semd):
    c = lax.axis_index("c")
    s = lax.axis_index("s")
    iot = lax.iota(jnp.int32, 16)
    z16 = jnp.zeros((16,), jnp.float32)

    def zero_rows0():
        def zb(r, _):
            rows[0, r, pl.ds(0, 16)] = z16
            rows[0, r, pl.ds(16, 16)] = z16
            return 0
        lax.fori_loop(0, _DR, zb, 0)

    def fire_idx(jc, b):
        eb = s * _ET + jc * _CH
        pltpu.async_copy(src_ref.at[pl.ds(eb, _CH)], si.at[b], semi.at[b])
        pltpu.async_copy(dst_ref.at[pl.ds(eb, _CH)], di.at[b], semi.at[b])

    zsrc = rows.at[0, pl.ds(0, _DR)]
    for p in range(2):
        chunk = 2 * p + c

        if p == 0:
            zero_rows0()

            def zacc(k, _):
                pltpu.sync_copy(zsrc, acc.at[pl.ds(s * _TRW + k * _DR, _DR)])
                return 0
            lax.fori_loop(0, _TRW // _DR, zacc, 0)
        plsc.subcore_barrier()

        for b in range(_NB):
            fire_idx(b, b)

        def group(g, _):
            gds = []
            for b in range(_NB):
                pltpu.make_async_copy(src_ref.at[pl.ds(0, _CH)],
                                      si.at[b], semi.at[b]).wait()
                pltpu.make_async_copy(dst_ref.at[pl.ds(0, _CH)],
                                      di.at[b], semi.at[b]).wait()
                for gg in range(_CH // 16):
                    sl = pl.ds(gg * 16, 16)
                    gi[b, sl] = si[b, sl] * 4 + chunk
                gds.append(pltpu.async_copy(ego_ref.at[gi.at[b]],
                                            rows.at[b], semg.at[b]))
            sds = []
            for b in range(_NB):
                gds[b].wait()
                sds.append(pltpu.async_copy(rows.at[b], acc.at[di.at[b]],
                                            sems.at[b], add=True))
            for b in range(_NB):
                sds[b].wait()

                @pl.when(g < _NG - 1)
                def _():
                    fire_idx((g + 1) * _NB + b, b)
            return 0
        lax.fori_loop(0, _NG, group, 0)
        plsc.subcore_barrier()

        if p == 0:
            zero_rows0()

        def dbody(k, _):
            rb = s * _TRW + k * _DR
            dsl = rows.at[1, pl.ds(0, _DR)]
            pltpu.sync_copy(acc.at[pl.ds(rb, _DR)], dsl)
            if p == 0:
                pltpu.sync_copy(zsrc, acc.at[pl.ds(rb, _DR)])
            for gg in range(_DR // 16):
                didx[pl.ds(gg * 16, 16)] = 4 * rb + 64 * gg + 4 * iot + chunk
            pltpu.async_copy(dsl, out_ref.at[didx], semd).wait()
            return 0
        lax.fori_loop(0, _TRW // _DR, dbody, 0)


def _scatter_layer(ep, srcp, dstp):
    ego_v = ep.reshape(RT * 4, 32)
    run = functools.partial(
        pl.kernel,
        out_type=jax.ShapeDtypeStruct((RT * 4, 32), jnp.float32),
        mesh=_sc_mesh(),
        compiler_params=pltpu.CompilerParams(use_tc_tiling_on_sc=False),
        scratch_types=[
            pltpu.VMEM_SHARED((RT, 32), jnp.float32),
            pltpu.VMEM((_NB, _CH), jnp.int32),
            pltpu.VMEM((_NB, _CH), jnp.int32),
            pltpu.VMEM((_NB, _CH), jnp.int32),
            pltpu.VMEM((_NB, _CH, 32), jnp.float32),
            pltpu.VMEM((_DR,), jnp.int32),
            pltpu.SemaphoreType.DMA((_NB,)),
            pltpu.SemaphoreType.DMA((_NB,)),
            pltpu.SemaphoreType.DMA((_NB,)),
            pltpu.SemaphoreType.DMA,
        ],
    )(_sc_scatter_body)
    return run(ego_v, srcp, dstp).reshape(RT, HP)


_GW = (B * T) // (_NC * _NS)


def _sc_gather_body(tab_ref, gi_ref, out_ref, i128, i16, r128, r16, sem):
    cc = lax.axis_index("c")
    ss = lax.axis_index("s")
    base = (ss * _NC + cc) * _GW
    for off in (0, 128, 256):
        pltpu.sync_copy(gi_ref.at[pl.ds(base + off, 128)], i128)
        pltpu.async_copy(tab_ref.at[i128], r128, sem).wait()
        pltpu.sync_copy(r128, out_ref.at[pl.ds(base + off, 128)])
    pltpu.sync_copy(gi_ref.at[pl.ds(base + 384, 16)], i16)
    pltpu.async_copy(tab_ref.at[i16], r16, sem).wait()
    pltpu.sync_copy(r16, out_ref.at[pl.ds(base + 384, 16)])


def _session_gather(finalp, gidx):
    run = functools.partial(
        pl.kernel,
        out_type=jax.ShapeDtypeStruct((B * T, HP), jnp.float32),
        mesh=_sc_mesh(),
        scratch_types=[
            pltpu.VMEM((128,), jnp.int32),
            pltpu.VMEM((16,), jnp.int32),
            pltpu.VMEM((128, HP), jnp.float32),
            pltpu.VMEM((16, HP), jnp.float32),
            pltpu.SemaphoreType.DMA,
        ],
    )(_sc_gather_body)
    return run(finalp, gidx)


def _pad_w(w):
    return jnp.pad(w, ((0, HP - w.shape[0]), (0, HP - w.shape[1])))


def _pad_b(b):
    return jnp.pad(b.reshape(1, -1), ((0, 0), (0, HP - b.shape[-1])))


@jax.jit
def kernel(edge_index, item, alias, mask, tar, embedding,
           W_gc_0, b_gc_0, W_gc_1, b_gc_1, W_concat, b_concat,
           nasr_w2, nasr_w3, nasr_v, nasr_b):
    npad = _EPAD - E
    srcp = jnp.concatenate([edge_index[0],
                            (jnp.arange(npad, dtype=jnp.int32) * 131) % N])
    dstp = jnp.concatenate([edge_index[1],
                            N + (jnp.arange(npad, dtype=jnp.int32) % (RT - N))])

    e0p = jnp.pad(embedding, ((0, RT - N), (0, HP - H)))
    w0p = _pad_w(W_gc_0)
    b0p = _pad_b(b_gc_0)
    w1p = _pad_w(W_gc_1)
    b1p = _pad_b(b_gc_1)
    wc0p = _pad_w(W_concat[:H])
    wc1p = _pad_w(W_concat[H:2 * H])
    wc2p = _pad_w(W_concat[2 * H:])
    bcp = _pad_b(b_concat)
    w2p = _pad_w(nasr_w2)
    w3p = _pad_w(nasr_w3)
    vpp = _pad_b(nasr_v[0])
    nbp = _pad_b(nasr_b)

    agg1p = _scatter_layer(e0p, srcp, dstp)
    e1p = _mm_act(agg1p, w0p, b0p)
    agg2p = _scatter_layer(e1p, srcp, dstp)
    finalp, ssq_e0 = _final_stage(agg2p, e0p, e1p, w1p, b1p,
                                  wc0p, wc1p, wc2p, bcp)

    gidx = jnp.take_along_axis(item, alias, axis=1)
    seq_hp = _session_gather(finalp, gidx.reshape(-1))
    last_h = seq_hp.reshape(B, T, HP)[:, T - 1]
    valid = (item != 0).astype(jnp.float32)
    maskf = mask.reshape(B * T, 1)
    etp = finalp[:T]

    map_ = _session_stage(seq_hp, last_h, valid, maskf, etp,
                          w2p, w3p, vpp, nbp)

    labels = jnp.clip(tar - 1, 0, N - 2).astype(jnp.int32).reshape(B, 1)
    btabp = e0p[1:N]
    logits, rmax, rsum, rlab = _logits_stage(map_, btabp, labels)

    lse = jnp.log(rsum[:, 0]) + rmax[:, 0]
    ce = jnp.mean(lse - rlab[:, 0])
    wsum = (jnp.sum(W_gc_0 ** 2) + jnp.sum(b_gc_0 ** 2)
            + jnp.sum(W_gc_1 ** 2) + jnp.sum(b_gc_1 ** 2)
            + jnp.sum(W_concat ** 2) + jnp.sum(b_concat ** 2)
            + jnp.sum(nasr_w2 ** 2) + jnp.sum(nasr_w3 ** 2)
            + jnp.sum(nasr_v ** 2) + jnp.sum(nasr_b ** 2))
    l2 = 0.5 * (ssq_e0[0, 0] + wsum) * L2C
    loss = ce + l2
    return (loss, logits)


# speedup vs baseline: 7.2345x; 1.0247x over previous
import functools

import jax
import jax.numpy as jnp
from jax import lax
from jax.experimental import pallas as pl
from jax.experimental.pallas import tpu as pltpu
from jax.experimental.pallas import tpu_sc as plsc

N = 50000
H = 100
HP = 128
B = 256
T = 50
E = 800000
V = N - 1
L2C = 1e-05

RT = 50176
RB = 3136
CB = 2048


def _leaky(x):
    return jnp.where(x >= 0, x, 0.2 * x)


def _mm_act_body(x_ref, w_ref, b_ref, o_ref):
    o_ref[...] = _leaky(
        jnp.dot(x_ref[...], w_ref[...], preferred_element_type=jnp.float32, precision=lax.Precision.HIGHEST)
        + b_ref[...])


def _mm_act(x, w, b):
    return pl.pallas_call(
        _mm_act_body,
        grid=(RT // RB,),
        in_specs=[
            pl.BlockSpec((RB, HP), lambda i: (i, 0)),
            pl.BlockSpec((HP, HP), lambda i: (0, 0)),
            pl.BlockSpec((1, HP), lambda i: (0, 0)),
        ],
        out_specs=pl.BlockSpec((RB, HP), lambda i: (i, 0)),
        out_shape=jax.ShapeDtypeStruct((RT, HP), jnp.float32),
    )(x, w, b)


def _final_body(agg2_ref, e0_ref, e1_ref, w1_ref, b1_ref,
                wc0_ref, wc1_ref, wc2_ref, bc_ref, o_ref, ssq_ref):
    i = pl.program_id(0)
    e0 = e0_ref[...]
    e2 = _leaky(
        jnp.dot(agg2_ref[...], w1_ref[...], preferred_element_type=jnp.float32, precision=lax.Precision.HIGHEST)
        + b1_ref[...])
    f = (jnp.dot(e0, wc0_ref[...], preferred_element_type=jnp.float32, precision=lax.Precision.HIGHEST)
         + jnp.dot(e1_ref[...], wc1_ref[...], preferred_element_type=jnp.float32, precision=lax.Precision.HIGHEST)
         + jnp.dot(e2, wc2_ref[...], preferred_element_type=jnp.float32, precision=lax.Precision.HIGHEST)
         + bc_ref[...])
    o_ref[...] = _leaky(f)
    part = jnp.sum(e0 * e0).reshape(1, 1)

    @pl.when(i == 0)
    def _():
        ssq_ref[...] = part

    @pl.when(i > 0)
    def _():
        ssq_ref[...] += part


def _final_stage(agg2p, e0p, e1p, w1, b1, wc0, wc1, wc2, bc):
    return pl.pallas_call(
        _final_body,
        grid=(RT // RB,),
        in_specs=[
            pl.BlockSpec((RB, HP), lambda i: (i, 0)),
            pl.BlockSpec((RB, HP), lambda i: (i, 0)),
            pl.BlockSpec((RB, HP), lambda i: (i, 0)),
        ] + [pl.BlockSpec((HP, HP), lambda i: (0, 0))] * 1
          + [pl.BlockSpec((1, HP), lambda i: (0, 0))]
          + [pl.BlockSpec((HP, HP), lambda i: (0, 0))] * 3
          + [pl.BlockSpec((1, HP), lambda i: (0, 0))],
        out_specs=[
            pl.BlockSpec((RB, HP), lambda i: (i, 0)),
            pl.BlockSpec((1, 1), lambda i: (0, 0)),
        ],
        out_shape=[
            jax.ShapeDtypeStruct((RT, HP), jnp.float32),
            jax.ShapeDtypeStruct((1, 1), jnp.float32),
        ],
    )(agg2p, e0p, e1p, w1, b1, wc0, wc1, wc2, bc)


_SB = 32
_XB = _SB * T


def _session_body(x_ref, last_ref, valid_ref, maskf_ref, et_ref,
                  w2_ref, w3_ref, v_ref, nb_ref, ma_ref):
    x = x_ref[...]
    lasth = last_ref[...]
    ri = lax.broadcasted_iota(jnp.int32, (_XB, _SB), 0) // T
    ci = lax.broadcasted_iota(jnp.int32, (_XB, _SB), 1)
    sel = (ri == ci).astype(jnp.float32)
    lastb = jnp.dot(sel, lasth, preferred_element_type=jnp.float32, precision=lax.Precision.HIGHEST)
    seq = jnp.dot(x, w2_ref[...], preferred_element_type=jnp.float32, precision=lax.Precision.HIGHEST)
    m = jax.nn.sigmoid(lastb + seq + nb_ref[...])
    coef = jnp.dot(m, v_ref[...].T, preferred_element_type=jnp.float32, precision=lax.Precision.HIGHEST)
    coef = coef * maskf_ref[...]
    glb = lax.dot_general(sel, coef * x, (((0,), (0,)), ((), ())),
                          preferred_element_type=jnp.float32, precision=lax.Precision.HIGHEST)
    vald = valid_ref[...]
    cnt = jnp.clip(jnp.sum(vald, axis=1, keepdims=True), 1.0, None)
    last_emb = jnp.dot(vald, et_ref[...], preferred_element_type=jnp.float32, precision=lax.Precision.HIGHEST) / cnt
    gate = glb + jnp.dot(last_emb, w3_ref[...], preferred_element_type=jnp.float32, precision=lax.Precision.HIGHEST)
    ma_ref[...] = gate * lasth


def _session_stage(seq_hp, last_h, valid, maskf, etp, w2, w3, vp, nbp):
    return pl.pallas_call(
        _session_body,
        grid=(B // _SB,),
        in_specs=[
            pl.BlockSpec((_XB, HP), lambda i: (i, 0)),
            pl.BlockSpec((_SB, HP), lambda i: (i, 0)),
            pl.BlockSpec((_SB, T), lambda i: (i, 0)),
            pl.BlockSpec((_XB, 1), lambda i: (i, 0)),
            pl.BlockSpec((T, HP), lambda i: (0, 0)),
            pl.BlockSpec((HP, HP), lambda i: (0, 0)),
            pl.BlockSpec((HP, HP), lambda i: (0, 0)),
            pl.BlockSpec((1, HP), lambda i: (0, 0)),
            pl.BlockSpec((1, HP), lambda i: (0, 0)),
        ],
        out_specs=pl.BlockSpec((_SB, HP), lambda i: (i, 0)),
        out_shape=jax.ShapeDtypeStruct((B, HP), jnp.float32),
    )(seq_hp, last_h, valid, maskf, etp, w2, w3, vp, nbp)


def _pad_body(x_ref, o_ref):
    i = pl.program_id(0)
    rowv = i * RB + lax.broadcasted_iota(jnp.int32, (RB, H), 0) < N
    x = jnp.where(rowv, x_ref[...], 0.0)
    o_ref[...] = jnp.concatenate([x, jnp.zeros((RB, HP - H), jnp.float32)],
                                 axis=1)


def _pad_stage(embedding):
    return pl.pallas_call(
        _pad_body,
        grid=(RT // RB,),
        in_specs=[pl.BlockSpec((RB, H), lambda i: (i, 0))],
        out_specs=pl.BlockSpec((RB, HP), lambda i: (i, 0)),
        out_shape=jax.ShapeDtypeStruct((RT, HP), jnp.float32),
    )(embedding)


def _logits_body(ma_ref, bt_ref, lab_ref, o_ref, mx_ref, sm_ref, lg_ref):
    i = pl.program_id(0)
    bt = bt_ref[...]
    lg = lax.dot_general(ma_ref[...], bt, (((1,), (1,)), ((), ())),
                         preferred_element_type=jnp.float32, precision=lax.Precision.HIGHEST)
    col = i * CB + lax.broadcasted_iota(jnp.int32, (B, CB), 1)
    okc = (col >= 1) & (col < N)
    lgm = jnp.where(okc, lg, -1e30)
    o_ref[...] = lg
    bm = jnp.max(lgm, axis=1, keepdims=True)
    bs = jnp.sum(jnp.exp(lgm - bm), axis=1, keepdims=True)
    labs = lab_ref[...]
    hit = jnp.where((col == labs) & okc, lg, 0.0)
    lab_part = jnp.sum(hit, axis=1, keepdims=True)

    @pl.when(i == 0)
    def _():
        mx_ref[...] = bm
        sm_ref[...] = bs
        lg_ref[...] = lab_part

    @pl.when(i > 0)
    def _():
        m_old = mx_ref[...]
        m_new = jnp.maximum(m_old, bm)
        sm_ref[...] = (sm_ref[...] * jnp.exp(m_old - m_new)
                       + bs * jnp.exp(bm - m_new))
        mx_ref[...] = m_new
        lg_ref[...] += lab_part


def _logits_stage(map_, e0p, labels1):
    nblk = pl.cdiv(N, CB)
    return pl.pallas_call(
        _logits_body,
        grid=(nblk,),
        in_specs=[
            pl.BlockSpec((B, HP), lambda i: (0, 0)),
            pl.BlockSpec((CB, HP), lambda i: (i, 0)),
            pl.BlockSpec((B, 1), lambda i: (0, 0)),
        ],
        out_specs=[
            pl.BlockSpec((B, CB), lambda i: (0, i)),
            pl.BlockSpec((B, 1), lambda i: (0, 0)),
            pl.BlockSpec((B, 1), lambda i: (0, 0)),
            pl.BlockSpec((B, 1), lambda i: (0, 0)),
        ],
        out_shape=[
            jax.ShapeDtypeStruct((B, N), jnp.float32),
            jax.ShapeDtypeStruct((B, 1), jnp.float32),
            jax.ShapeDtypeStruct((B, 1), jnp.float32),
            jax.ShapeDtypeStruct((B, 1), jnp.float32),
        ],
    )(map_, e0p, labels1)


_NC, _NS = 2, 16
_CH = 128
_EPAD = 811008
_ET = _EPAD // _NS
_NCHUNK = _ET // _CH
_TRW = RT // _NS
_DR = 112


def _sc_mesh():
    return plsc.VectorSubcoreMesh(core_axis_name="c", subcore_axis_name="s",
                                  num_cores=_NC, num_subcores=_NS)


_NB = 6
_NG = _NCHUNK // _NB


def _sc_scatter_body(ego_ref, src_ref, dst_ref, out_ref,
                     acc, si, di, gi, rows, didx,
                     semi, semg, sems, semd):
    c = lax.axis_index("c")
    s = lax.axis_index("s")
    iot = lax.iota(jnp.int32, 16)
    z16 = jnp.zeros((16,), jnp.float32)

    def zero_rows0():
        def zb(r, _):
            rows[0, r, pl.ds(0, 16)] = z16
            rows[0, r, pl.ds(16, 16)] = z16
            return 0
        lax.fori_loop(0, _DR, zb, 0)

    def fire_idx(jc, b):
        eb = s * _ET + jc * _CH
        pltpu.async_copy(src_ref.at[pl.ds(eb, _CH)], si.at[b], semi.at[b])
        pltpu.async_copy(dst_ref.at[pl.ds(eb, _CH)], di.at[b], semi.at[b])

    zsrc = rows.at[0, pl.ds(0, _DR)]
    for p in range(2):
        chunk = 2 * p + c

        if p == 0:
            zero_rows0()

            def zacc(k, _):
                pltpu.sync_copy(zsrc, acc.at[pl.ds(s * _TRW + k * _DR, _DR)])
                return 0
            lax.fori_loop(0, _TRW // _DR, zacc, 0)
        plsc.subcore_barrier()

        for b in range(_NB):
            fire_idx(b, b)

        def group(g, _):
            gds = []
            for b in range(_NB):
                pltpu.make_async_copy(src_ref.at[pl.ds(0, _CH)],
                                      si.at[b], semi.at[b]).wait()
                pltpu.make_async_copy(dst_ref.at[pl.ds(0, _CH)],
                                      di.at[b], semi.at[b]).wait()
                for gg in range(_CH // 16):
                    sl = pl.ds(gg * 16, 16)
                    gi[b, sl] = si[b, sl] * 4 + chunk
                gds.append(pltpu.async_copy(ego_ref.at[gi.at[b]],
                                            rows.at[b], semg.at[b]))
            sds = []
            for b in range(_NB):
                gds[b].wait()
                sds.append(pltpu.async_copy(rows.at[b], acc.at[di.at[b]],
                                            sems.at[b], add=True))
            for b in range(_NB):
                sds[b].wait()

                @pl.when(g < _NG - 1)
                def _():
                    fire_idx((g + 1) * _NB + b, b)
            return 0
        lax.fori_loop(0, _NG, group, 0)
        plsc.subcore_barrier()

        if p == 0:
            zero_rows0()

        def dbody(k, _):
            rb = s * _TRW + k * _DR
            dsl = rows.at[1, pl.ds(0, _DR)]
            pltpu.sync_copy(acc.at[pl.ds(rb, _DR)], dsl)
            if p == 0:
                pltpu.sync_copy(zsrc, acc.at[pl.ds(rb, _DR)])
            for gg in range(_DR // 16):
                didx[pl.ds(gg * 16, 16)] = 4 * rb + 64 * gg + 4 * iot + chunk
            pltpu.async_copy(dsl, out_ref.at[didx], semd).wait()
            return 0
        lax.fori_loop(0, _TRW // _DR, dbody, 0)


def _scatter_layer(ep, srcp, dstp):
    ego_v = ep.reshape(RT * 4, 32)
    run = functools.partial(
        pl.kernel,
        out_type=jax.ShapeDtypeStruct((RT * 4, 32), jnp.float32),
        mesh=_sc_mesh(),
        compiler_params=pltpu.CompilerParams(use_tc_tiling_on_sc=False),
        scratch_types=[
            pltpu.VMEM_SHARED((RT, 32), jnp.float32),
            pltpu.VMEM((_NB, _CH), jnp.int32),
            pltpu.VMEM((_NB, _CH), jnp.int32),
            pltpu.VMEM((_NB, _CH), jnp.int32),
            pltpu.VMEM((_NB, _CH, 32), jnp.float32),
            pltpu.VMEM((_DR,), jnp.int32),
            pltpu.SemaphoreType.DMA((_NB,)),
            pltpu.SemaphoreType.DMA((_NB,)),
            pltpu.SemaphoreType.DMA((_NB,)),
            pltpu.SemaphoreType.DMA,
        ],
    )(_sc_scatter_body)
    return run(ego_v, srcp, dstp).reshape(RT, HP)


_GW = (B * T) // (_NC * _NS)


def _sc_gather_body(tab_ref, gi_ref, out_ref, i128, i16, r128, r16, sem):
    cc = lax.axis_index("c")
    ss = lax.axis_index("s")
    base = (ss * _NC + cc) * _GW
    for off in (0, 128, 256):
        pltpu.sync_copy(gi_ref.at[pl.ds(base + off, 128)], i128)
        pltpu.async_copy(tab_ref.at[i128], r128, sem).wait()
        pltpu.sync_copy(r128, out_ref.at[pl.ds(base + off, 128)])
    pltpu.sync_copy(gi_ref.at[pl.ds(base + 384, 16)], i16)
    pltpu.async_copy(tab_ref.at[i16], r16, sem).wait()
    pltpu.sync_copy(r16, out_ref.at[pl.ds(base + 384, 16)])


def _session_gather(finalp, gidx):
    run = functools.partial(
        pl.kernel,
        out_type=jax.ShapeDtypeStruct((B * T, HP), jnp.float32),
        mesh=_sc_mesh(),
        scratch_types=[
            pltpu.VMEM((128,), jnp.int32),
            pltpu.VMEM((16,), jnp.int32),
            pltpu.VMEM((128, HP), jnp.float32),
            pltpu.VMEM((16, HP), jnp.float32),
            pltpu.SemaphoreType.DMA,
        ],
    )(_sc_gather_body)
    return run(finalp, gidx)


def _pad_w(w):
    return jnp.pad(w, ((0, HP - w.shape[0]), (0, HP - w.shape[1])))


def _pad_b(b):
    return jnp.pad(b.reshape(1, -1), ((0, 0), (0, HP - b.shape[-1])))


@jax.jit
def kernel(edge_index, item, alias, mask, tar, embedding,
           W_gc_0, b_gc_0, W_gc_1, b_gc_1, W_concat, b_concat,
           nasr_w2, nasr_w3, nasr_v, nasr_b):
    npad = _EPAD - E
    srcp = jnp.concatenate([edge_index[0],
                            (jnp.arange(npad, dtype=jnp.int32) * 131) % N])
    dstp = jnp.concatenate([edge_index[1],
                            N + (jnp.arange(npad, dtype=jnp.int32) % (RT - N))])

    e0p = _pad_stage(embedding)
    w0p = _pad_w(W_gc_0)
    b0p = _pad_b(b_gc_0)
    w1p = _pad_w(W_gc_1)
    b1p = _pad_b(b_gc_1)
    wc0p = _pad_w(W_concat[:H])
    wc1p = _pad_w(W_concat[H:2 * H])
    wc2p = _pad_w(W_concat[2 * H:])
    bcp = _pad_b(b_concat)
    w2p = _pad_w(nasr_w2)
    w3p = _pad_w(nasr_w3)
    vpp = _pad_b(nasr_v[0])
    nbp = _pad_b(nasr_b)

    agg1p = _scatter_layer(e0p, srcp, dstp)
    e1p = _mm_act(agg1p, w0p, b0p)
    agg2p = _scatter_layer(e1p, srcp, dstp)
    finalp, ssq_e0 = _final_stage(agg2p, e0p, e1p, w1p, b1p,
                                  wc0p, wc1p, wc2p, bcp)

    gidx = jnp.take_along_axis(item, alias, axis=1)
    seq_hp = _session_gather(finalp, gidx.reshape(-1))
    last_h = seq_hp.reshape(B, T, HP)[:, T - 1]
    valid = (item != 0).astype(jnp.float32)
    maskf = mask.reshape(B * T, 1)
    etp = finalp[:T]

    map_ = _session_stage(seq_hp, last_h, valid, maskf, etp,
                          w2p, w3p, vpp, nbp)

    labels1 = (jnp.clip(tar - 1, 0, N - 2) + 1).astype(jnp.int32).reshape(B, 1)
    logits_full, rmax, rsum, rlab = _logits_stage(map_, e0p, labels1)
    logits = logits_full[:, 1:]

    lse = jnp.log(rsum[:, 0]) + rmax[:, 0]
    ce = jnp.mean(lse - rlab[:, 0])
    wsum = (jnp.sum(W_gc_0 ** 2) + jnp.sum(b_gc_0 ** 2)
            + jnp.sum(W_gc_1 ** 2) + jnp.sum(b_gc_1 ** 2)
            + jnp.sum(W_concat ** 2) + jnp.sum(b_concat ** 2)
            + jnp.sum(nasr_w2 ** 2) + jnp.sum(nasr_w3 ** 2)
            + jnp.sum(nasr_v ** 2) + jnp.sum(nasr_b ** 2))
    l2 = 0.5 * (ssq_e0[0, 0] + wsum) * L2C
    loss = ce + l2
    return (loss, logits)


# speedup vs baseline: 7.3736x; 1.0192x over previous
import functools

import jax
import jax.numpy as jnp
from jax import lax
from jax.experimental import pallas as pl
from jax.experimental.pallas import tpu as pltpu
from jax.experimental.pallas import tpu_sc as plsc

N = 50000
H = 100
HP = 128
B = 256
T = 50
E = 800000
V = N - 1
L2C = 1e-05

RT = 50176
RB = 3136
CB = 2048


def _leaky(x):
    return jnp.where(x >= 0, x, 0.2 * x)


def _mm_act_body(x_ref, w_ref, b_ref, o_ref):
    o_ref[...] = _leaky(
        jnp.dot(x_ref[...], w_ref[...], preferred_element_type=jnp.float32, precision=lax.Precision.HIGHEST)
        + b_ref[...])


def _mm_act(x, w, b):
    return pl.pallas_call(
        _mm_act_body,
        grid=(RT // RB,),
        in_specs=[
            pl.BlockSpec((RB, HP), lambda i: (i, 0)),
            pl.BlockSpec((HP, HP), lambda i: (0, 0)),
            pl.BlockSpec((1, HP), lambda i: (0, 0)),
        ],
        out_specs=pl.BlockSpec((RB, HP), lambda i: (i, 0)),
        out_shape=jax.ShapeDtypeStruct((RT, HP), jnp.float32),
    )(x, w, b)


def _final_body(agg2_ref, e0_ref, e1_ref, w1_ref, b1_ref,
                wc0_ref, wc1_ref, wc2_ref, bc_ref, o_ref, ssq_ref):
    i = pl.program_id(0)
    e0 = e0_ref[...]
    e2 = _leaky(
        jnp.dot(agg2_ref[...], w1_ref[...], preferred_element_type=jnp.float32, precision=lax.Precision.HIGHEST)
        + b1_ref[...])
    f = (jnp.dot(e0, wc0_ref[...], preferred_element_type=jnp.float32, precision=lax.Precision.HIGHEST)
         + jnp.dot(e1_ref[...], wc1_ref[...], preferred_element_type=jnp.float32, precision=lax.Precision.HIGHEST)
         + jnp.dot(e2, wc2_ref[...], preferred_element_type=jnp.float32, precision=lax.Precision.HIGHEST)
         + bc_ref[...])
    o_ref[...] = _leaky(f)
    part = jnp.sum(e0 * e0).reshape(1, 1)

    @pl.when(i == 0)
    def _():
        ssq_ref[...] = part

    @pl.when(i > 0)
    def _():
        ssq_ref[...] += part


def _final_stage(agg2p, e0p, e1p, w1, b1, wc0, wc1, wc2, bc):
    return pl.pallas_call(
        _final_body,
        grid=(RT // RB,),
        in_specs=[
            pl.BlockSpec((RB, HP), lambda i: (i, 0)),
            pl.BlockSpec((RB, HP), lambda i: (i, 0)),
            pl.BlockSpec((RB, HP), lambda i: (i, 0)),
        ] + [pl.BlockSpec((HP, HP), lambda i: (0, 0))] * 1
          + [pl.BlockSpec((1, HP), lambda i: (0, 0))]
          + [pl.BlockSpec((HP, HP), lambda i: (0, 0))] * 3
          + [pl.BlockSpec((1, HP), lambda i: (0, 0))],
        out_specs=[
            pl.BlockSpec((RB, HP), lambda i: (i, 0)),
            pl.BlockSpec((1, 1), lambda i: (0, 0)),
        ],
        out_shape=[
            jax.ShapeDtypeStruct((RT, HP), jnp.float32),
            jax.ShapeDtypeStruct((1, 1), jnp.float32),
        ],
    )(agg2p, e0p, e1p, w1, b1, wc0, wc1, wc2, bc)


_SB = 32
_XB = _SB * T


def _session_body(x_ref, last_ref, valid_ref, maskf_ref, et_ref,
                  w2_ref, w3_ref, v_ref, nb_ref, ma_ref):
    x = x_ref[...]
    lasth = last_ref[...]
    ri = lax.broadcasted_iota(jnp.int32, (_XB, _SB), 0) // T
    ci = lax.broadcasted_iota(jnp.int32, (_XB, _SB), 1)
    sel = (ri == ci).astype(jnp.float32)
    lastb = jnp.dot(sel, lasth, preferred_element_type=jnp.float32, precision=lax.Precision.HIGHEST)
    seq = jnp.dot(x, w2_ref[...], preferred_element_type=jnp.float32, precision=lax.Precision.HIGHEST)
    m = jax.nn.sigmoid(lastb + seq + nb_ref[...])
    coef = jnp.dot(m, v_ref[...].T, preferred_element_type=jnp.float32, precision=lax.Precision.HIGHEST)
    coef = coef * maskf_ref[...]
    glb = lax.dot_general(sel, coef * x, (((0,), (0,)), ((), ())),
                          preferred_element_type=jnp.float32, precision=lax.Precision.HIGHEST)
    vald = valid_ref[...]
    cnt = jnp.clip(jnp.sum(vald, axis=1, keepdims=True), 1.0, None)
    last_emb = jnp.dot(vald, et_ref[...], preferred_element_type=jnp.float32, precision=lax.Precision.HIGHEST) / cnt
    gate = glb + jnp.dot(last_emb, w3_ref[...], preferred_element_type=jnp.float32, precision=lax.Precision.HIGHEST)
    ma_ref[...] = gate * lasth


def _session_stage(seq_hp, last_h, valid, maskf, etp, w2, w3, vp, nbp):
    return pl.pallas_call(
        _session_body,
        grid=(B // _SB,),
        in_specs=[
            pl.BlockSpec((_XB, HP), lambda i: (i, 0)),
            pl.BlockSpec((_SB, HP), lambda i: (i, 0)),
            pl.BlockSpec((_SB, T), lambda i: (i, 0)),
            pl.BlockSpec((_XB, 1), lambda i: (i, 0)),
            pl.BlockSpec((T, HP), lambda i: (0, 0)),
            pl.BlockSpec((HP, HP), lambda i: (0, 0)),
            pl.BlockSpec((HP, HP), lambda i: (0, 0)),
            pl.BlockSpec((1, HP), lambda i: (0, 0)),
            pl.BlockSpec((1, HP), lambda i: (0, 0)),
        ],
        out_specs=pl.BlockSpec((_SB, HP), lambda i: (i, 0)),
        out_shape=jax.ShapeDtypeStruct((B, HP), jnp.float32),
    )(seq_hp, last_h, valid, maskf, etp, w2, w3, vp, nbp)


def _pad_body(x_ref, o_ref):
    i = pl.program_id(0)
    rowv = i * RB + lax.broadcasted_iota(jnp.int32, (RB, H), 0) < N
    x = jnp.where(rowv, x_ref[...], 0.0)
    o_ref[...] = jnp.concatenate([x, jnp.zeros((RB, HP - H), jnp.float32)],
                                 axis=1)


def _pad_stage(embedding):
    return pl.pallas_call(
        _pad_body,
        grid=(RT // RB,),
        in_specs=[pl.BlockSpec((RB, H), lambda i: (i, 0))],
        out_specs=pl.BlockSpec((RB, HP), lambda i: (i, 0)),
        out_shape=jax.ShapeDtypeStruct((RT, HP), jnp.float32),
    )(embedding)


def _logits_body(ma_ref, bt_ref, lab_ref, o_ref, mx_ref, sm_ref, lg_ref):
    i = pl.program_id(0)
    bt = bt_ref[...]
    lg = lax.dot_general(ma_ref[...], bt, (((1,), (1,)), ((), ())),
                         preferred_element_type=jnp.float32, precision=lax.Precision.HIGHEST)
    col = i * CB + lax.broadcasted_iota(jnp.int32, (B, CB), 1)
    okc = (col >= 1) & (col < N)
    lgm = jnp.where(okc, lg, -1e30)
    o_ref[...] = lg
    bm = jnp.max(lgm, axis=1, keepdims=True)
    bs = jnp.sum(jnp.exp(lgm - bm), axis=1, keepdims=True)
    labs = lab_ref[...]
    hit = jnp.where((col == labs) & okc, lg, 0.0)
    lab_part = jnp.sum(hit, axis=1, keepdims=True)

    @pl.when(i == 0)
    def _():
        mx_ref[...] = bm
        sm_ref[...] = bs
        lg_ref[...] = lab_part

    @pl.when(i > 0)
    def _():
        m_old = mx_ref[...]
        m_new = jnp.maximum(m_old, bm)
        sm_ref[...] = (sm_ref[...] * jnp.exp(m_old - m_new)
                       + bs * jnp.exp(bm - m_new))
        mx_ref[...] = m_new
        lg_ref[...] += lab_part


def _logits_stage(map_, e0p, labels1):
    nblk = pl.cdiv(N, CB)
    return pl.pallas_call(
        _logits_body,
        grid=(nblk,),
        in_specs=[
            pl.BlockSpec((B, HP), lambda i: (0, 0)),
            pl.BlockSpec((CB, HP), lambda i: (i, 0)),
            pl.BlockSpec((B, 1), lambda i: (0, 0)),
        ],
        out_specs=[
            pl.BlockSpec((B, CB), lambda i: (0, i)),
            pl.BlockSpec((B, 1), lambda i: (0, 0)),
            pl.BlockSpec((B, 1), lambda i: (0, 0)),
            pl.BlockSpec((B, 1), lambda i: (0, 0)),
        ],
        out_shape=[
            jax.ShapeDtypeStruct((B, N), jnp.float32),
            jax.ShapeDtypeStruct((B, 1), jnp.float32),
            jax.ShapeDtypeStruct((B, 1), jnp.float32),
            jax.ShapeDtypeStruct((B, 1), jnp.float32),
        ],
    )(map_, e0p, labels1)


_NC, _NS = 2, 16
_CH = 128
_EPAD = 811008
_ET = _EPAD // _NS
_NCHUNK = _ET // _CH
_TRW = RT // _NS
_DR = 112


def _sc_mesh():
    return plsc.VectorSubcoreMesh(core_axis_name="c", subcore_axis_name="s",
                                  num_cores=_NC, num_subcores=_NS)


_NB = 6
_NG = _NCHUNK // _NB


def _sc_scatter_body(ego_ref, src_ref, dst_ref, out_ref,
                     acc, si, di, gi, rows, didx,
                     semi, semg, sems, semd):
    c = lax.axis_index("c")
    s = lax.axis_index("s")
    iot = lax.iota(jnp.int32, 16)
    z16 = jnp.zeros((16,), jnp.float32)

    def zero_rows0():
        def zb(r, _):
            rows[0, r, pl.ds(0, 16)] = z16
            rows[0, r, pl.ds(16, 16)] = z16
            return 0
        lax.fori_loop(0, _DR, zb, 0)

    def fire_idx(jc, b):
        eb = s * _ET + jc * _CH
        pltpu.async_copy(src_ref.at[pl.ds(eb, _CH)], si.at[b], semi.at[b])
        pltpu.async_copy(dst_ref.at[pl.ds(eb, _CH)], di.at[b], semi.at[b])

    zsrc = rows.at[0, pl.ds(0, _DR)]
    for p in range(2):
        chunk = 2 * p + c

        if p == 0:
            for b in range(_NB):
                fire_idx(b, b)
            zero_rows0()

            def zacc(k, _):
                pltpu.sync_copy(zsrc, acc.at[pl.ds(s * _TRW + k * _DR, _DR)])
                return 0
            lax.fori_loop(0, _TRW // _DR, zacc, 0)
        plsc.subcore_barrier()

        def group(g, _):
            gds = []
            for b in range(_NB):
                pltpu.make_async_copy(src_ref.at[pl.ds(0, _CH)],
                                      si.at[b], semi.at[b]).wait()
                pltpu.make_async_copy(dst_ref.at[pl.ds(0, _CH)],
                                      di.at[b], semi.at[b]).wait()
                for gg in range(_CH // 16):
                    sl = pl.ds(gg * 16, 16)
                    gi[b, sl] = si[b, sl] * 4 + chunk
                gds.append(pltpu.async_copy(ego_ref.at[gi.at[b]],
                                            rows.at[b], semg.at[b]))
            sds = []
            for b in range(_NB):
                gds[b].wait()
                sds.append(pltpu.async_copy(rows.at[b], acc.at[di.at[b]],
                                            sems.at[b], add=True))
            for b in range(_NB):
                sds[b].wait()

                @pl.when(g < _NG - 1)
                def _():
                    fire_idx((g + 1) * _NB + b, b)
            return 0
        lax.fori_loop(0, _NG, group, 0)
        if p == 0:
            for b in range(_NB):
                fire_idx(b, b)
        plsc.subcore_barrier()

        if p == 0:
            zero_rows0()
        def dpair(k, _):
            for bb in range(2):
                kk = 2 * k + bb
                dsl = rows.at[1 + bb, pl.ds(0, _DR)]

                @pl.when(k > 0)
                def _():
                    pltpu.make_async_copy(dsl, out_ref.at[didx.at[bb]],
                                          semd.at[bb]).wait()
                rb = s * _TRW + kk * _DR
                pltpu.sync_copy(acc.at[pl.ds(rb, _DR)], dsl)
                if p == 0:
                    pltpu.sync_copy(zsrc, acc.at[pl.ds(rb, _DR)])
                for gg in range(_DR // 16):
                    didx[bb, pl.ds(gg * 16, 16)] = (4 * rb + 64 * gg
                                                    + 4 * iot + chunk)
                pltpu.async_copy(dsl, out_ref.at[didx.at[bb]], semd.at[bb])
            return 0
        lax.fori_loop(0, _TRW // _DR // 2, dpair, 0)
        for bb in range(2):
            pltpu.make_async_copy(rows.at[1 + bb, pl.ds(0, _DR)],
                                  out_ref.at[didx.at[bb]],
                                  semd.at[bb]).wait()


def _scatter_layer(ep, srcp, dstp):
    ego_v = ep.reshape(RT * 4, 32)
    run = functools.partial(
        pl.kernel,
        out_type=jax.ShapeDtypeStruct((RT * 4, 32), jnp.float32),
        mesh=_sc_mesh(),
        compiler_params=pltpu.CompilerParams(use_tc_tiling_on_sc=False),
        scratch_types=[
            pltpu.VMEM_SHARED((RT, 32), jnp.float32),
            pltpu.VMEM((_NB, _CH), jnp.int32),
            pltpu.VMEM((_NB, _CH), jnp.int32),
            pltpu.VMEM((_NB, _CH), jnp.int32),
            pltpu.VMEM((_NB, _CH, 32), jnp.float32),
            pltpu.VMEM((2, _DR), jnp.int32),
            pltpu.SemaphoreType.DMA((_NB,)),
            pltpu.SemaphoreType.DMA((_NB,)),
            pltpu.SemaphoreType.DMA((_NB,)),
            pltpu.SemaphoreType.DMA((2,)),
        ],
    )(_sc_scatter_body)
    return run(ego_v, srcp, dstp).reshape(RT, HP)


_GW = (B * T) // (_NC * _NS)


def _sc_gather_body(tab_ref, gi_ref, out_ref, i128, i16, r128, r16, sem):
    cc = lax.axis_index("c")
    ss = lax.axis_index("s")
    base = (ss * _NC + cc) * _GW
    for off in (0, 128, 256):
        pltpu.sync_copy(gi_ref.at[pl.ds(base + off, 128)], i128)
        pltpu.async_copy(tab_ref.at[i128], r128, sem).wait()
        pltpu.sync_copy(r128, out_ref.at[pl.ds(base + off, 128)])
    pltpu.sync_copy(gi_ref.at[pl.ds(base + 384, 16)], i16)
    pltpu.async_copy(tab_ref.at[i16], r16, sem).wait()
    pltpu.sync_copy(r16, out_ref.at[pl.ds(base + 384, 16)])


def _session_gather(finalp, gidx):
    run = functools.partial(
        pl.kernel,
        out_type=jax.ShapeDtypeStruct((B * T, HP), jnp.float32),
        mesh=_sc_mesh(),
        scratch_types=[
            pltpu.VMEM((128,), jnp.int32),
            pltpu.VMEM((16,), jnp.int32),
            pltpu.VMEM((128, HP), jnp.float32),
            pltpu.VMEM((16, HP), jnp.float32),
            pltpu.SemaphoreType.DMA,
        ],
    )(_sc_gather_body)
    return run(finalp, gidx)


def _pad_w(w):
    return jnp.pad(w, ((0, HP - w.shape[0]), (0, HP - w.shape[1])))


def _pad_b(b):
    return jnp.pad(b.reshape(1, -1), ((0, 0), (0, HP - b.shape[-1])))


@jax.jit
def kernel(edge_index, item, alias, mask, tar, embedding,
           W_gc_0, b_gc_0, W_gc_1, b_gc_1, W_concat, b_concat,
           nasr_w2, nasr_w3, nasr_v, nasr_b):
    npad = _EPAD - E
    srcp = jnp.concatenate([edge_index[0],
                            (jnp.arange(npad, dtype=jnp.int32) * 131) % N])
    dstp = jnp.concatenate([edge_index[1],
                            N + (jnp.arange(npad, dtype=jnp.int32) % (RT - N))])

    e0p = _pad_stage(embedding)
    w0p = _pad_w(W_gc_0)
    b0p = _pad_b(b_gc_0)
    w1p = _pad_w(W_gc_1)
    b1p = _pad_b(b_gc_1)
    wc0p = _pad_w(W_concat[:H])
    wc1p = _pad_w(W_concat[H:2 * H])
    wc2p = _pad_w(W_concat[2 * H:])
    bcp = _pad_b(b_concat)
    w2p = _pad_w(nasr_w2)
    w3p = _pad_w(nasr_w3)
    vpp = _pad_b(nasr_v[0])
    nbp = _pad_b(nasr_b)

    agg1p = _scatter_layer(e0p, srcp, dstp)
    e1p = _mm_act(agg1p, w0p, b0p)
    agg2p = _scatter_layer(e1p, srcp, dstp)
    finalp, ssq_e0 = _final_stage(agg2p, e0p, e1p, w1p, b1p,
                                  wc0p, wc1p, wc2p, bcp)

    gidx = jnp.take_along_axis(item, alias, axis=1)
    seq_hp = _session_gather(finalp, gidx.reshape(-1))
    last_h = seq_hp.reshape(B, T, HP)[:, T - 1]
    valid = (item != 0).astype(jnp.float32)
    maskf = mask.reshape(B * T, 1)
    etp = finalp[:T]

    map_ = _session_stage(seq_hp, last_h, valid, maskf, etp,
                          w2p, w3p, vpp, nbp)

    labels1 = (jnp.clip(tar - 1, 0, N - 2) + 1).astype(jnp.int32).reshape(B, 1)
    logits_full, rmax, rsum, rlab = _logits_stage(map_, e0p, labels1)
    logits = logits_full[:, 1:]

    lse = jnp.log(rsum[:, 0]) + rmax[:, 0]
    ce = jnp.mean(lse - rlab[:, 0])
    wsum = (jnp.sum(W_gc_0 ** 2) + jnp.sum(b_gc_0 ** 2)
            + jnp.sum(W_gc_1 ** 2) + jnp.sum(b_gc_1 ** 2)
            + jnp.sum(W_concat ** 2) + jnp.sum(b_concat ** 2)
            + jnp.sum(nasr_w2 ** 2) + jnp.sum(nasr_w3 ** 2)
            + jnp.sum(nasr_v ** 2) + jnp.sum(nasr_b ** 2))
    l2 = 0.5 * (ssq_e0[0, 0] + wsum) * L2C
    loss = ce + l2
    return (loss, logits)


# speedup vs baseline: 7.4517x; 1.0106x over previous
import functools

import jax
import jax.numpy as jnp
from jax import lax
from jax.experimental import pallas as pl
from jax.experimental.pallas import tpu as pltpu
from jax.experimental.pallas import tpu_sc as plsc

N = 50000
H = 100
HP = 128
B = 256
T = 50
E = 800000
V = N - 1
L2C = 1e-05

RT = 50176
RB = 3136
CB = 2048


def _leaky(x):
    return jnp.where(x >= 0, x, 0.2 * x)


def _mm_act_body(x_ref, w_ref, b_ref, o_ref):
    o_ref[...] = _leaky(
        jnp.dot(x_ref[...], w_ref[...], preferred_element_type=jnp.float32, precision=lax.Precision.HIGHEST)
        + b_ref[...])


def _mm_act(x, w, b):
    return pl.pallas_call(
        _mm_act_body,
        grid=(RT // RB,),
        in_specs=[
            pl.BlockSpec((RB, HP), lambda i: (i, 0)),
            pl.BlockSpec((HP, HP), lambda i: (0, 0)),
            pl.BlockSpec((1, HP), lambda i: (0, 0)),
        ],
        out_specs=pl.BlockSpec((RB, HP), lambda i: (i, 0)),
        out_shape=jax.ShapeDtypeStruct((RT, HP), jnp.float32),
    )(x, w, b)


def _final_body(agg2_ref, e0_ref, e1_ref, w1_ref, b1_ref,
                wc0_ref, wc1_ref, wc2_ref, bc_ref, o_ref, ssq_ref):
    i = pl.program_id(0)
    e0 = e0_ref[...]
    e2 = _leaky(
        jnp.dot(agg2_ref[...], w1_ref[...], preferred_element_type=jnp.float32, precision=lax.Precision.HIGHEST)
        + b1_ref[...])
    f = (jnp.dot(e0, wc0_ref[...], preferred_element_type=jnp.float32, precision=lax.Precision.HIGHEST)
         + jnp.dot(e1_ref[...], wc1_ref[...], preferred_element_type=jnp.float32, precision=lax.Precision.HIGHEST)
         + jnp.dot(e2, wc2_ref[...], preferred_element_type=jnp.float32, precision=lax.Precision.HIGHEST)
         + bc_ref[...])
    o_ref[...] = _leaky(f)
    part = jnp.sum(e0 * e0).reshape(1, 1)

    @pl.when(i == 0)
    def _():
        ssq_ref[...] = part

    @pl.when(i > 0)
    def _():
        ssq_ref[...] += part


def _final_stage(agg2p, e0p, e1p, w1, b1, wc0, wc1, wc2, bc):
    return pl.pallas_call(
        _final_body,
        grid=(RT // RB,),
        in_specs=[
            pl.BlockSpec((RB, HP), lambda i: (i, 0)),
            pl.BlockSpec((RB, HP), lambda i: (i, 0)),
            pl.BlockSpec((RB, HP), lambda i: (i, 0)),
        ] + [pl.BlockSpec((HP, HP), lambda i: (0, 0))] * 1
          + [pl.BlockSpec((1, HP), lambda i: (0, 0))]
          + [pl.BlockSpec((HP, HP), lambda i: (0, 0))] * 3
          + [pl.BlockSpec((1, HP), lambda i: (0, 0))],
        out_specs=[
            pl.BlockSpec((RB, HP), lambda i: (i, 0)),
            pl.BlockSpec((1, 1), lambda i: (0, 0)),
        ],
        out_shape=[
            jax.ShapeDtypeStruct((RT, HP), jnp.float32),
            jax.ShapeDtypeStruct((1, 1), jnp.float32),
        ],
    )(agg2p, e0p, e1p, w1, b1, wc0, wc1, wc2, bc)


_SB = 32
_XB = _SB * T


def _session_body(x_ref, last_ref, valid_ref, maskf_ref, et_ref,
                  w2_ref, w3_ref, v_ref, nb_ref, ma_ref):
    x = x_ref[...]
    lasth = last_ref[...]
    ri = lax.broadcasted_iota(jnp.int32, (_XB, _SB), 0) // T
    ci = lax.broadcasted_iota(jnp.int32, (_XB, _SB), 1)
    sel = (ri == ci).astype(jnp.float32)
    lastb = jnp.dot(sel, lasth, preferred_element_type=jnp.float32, precision=lax.Precision.HIGHEST)
    seq = jnp.dot(x, w2_ref[...], preferred_element_type=jnp.float32, precision=lax.Precision.HIGHEST)
    m = jax.nn.sigmoid(lastb + seq + nb_ref[...])
    coef = jnp.dot(m, v_ref[...].T, preferred_element_type=jnp.float32, precision=lax.Precision.HIGHEST)
    coef = coef * maskf_ref[...]
    glb = lax.dot_general(sel, coef * x, (((0,), (0,)), ((), ())),
                          preferred_element_type=jnp.float32, precision=lax.Precision.HIGHEST)
    vald = valid_ref[...]
    cnt = jnp.clip(jnp.sum(vald, axis=1, keepdims=True), 1.0, None)
    last_emb = jnp.dot(vald, et_ref[...], preferred_element_type=jnp.float32, precision=lax.Precision.HIGHEST) / cnt
    gate = glb + jnp.dot(last_emb, w3_ref[...], preferred_element_type=jnp.float32, precision=lax.Precision.HIGHEST)
    ma_ref[...] = gate * lasth


def _session_stage(seq_hp, last_h, valid, maskf, etp, w2, w3, vp, nbp):
    return pl.pallas_call(
        _session_body,
        grid=(B // _SB,),
        in_specs=[
            pl.BlockSpec((_XB, HP), lambda i: (i, 0)),
            pl.BlockSpec((_SB, HP), lambda i: (i, 0)),
            pl.BlockSpec((_SB, T), lambda i: (i, 0)),
            pl.BlockSpec((_XB, 1), lambda i: (i, 0)),
            pl.BlockSpec((T, HP), lambda i: (0, 0)),
            pl.BlockSpec((HP, HP), lambda i: (0, 0)),
            pl.BlockSpec((HP, HP), lambda i: (0, 0)),
            pl.BlockSpec((1, HP), lambda i: (0, 0)),
            pl.BlockSpec((1, HP), lambda i: (0, 0)),
        ],
        out_specs=pl.BlockSpec((_SB, HP), lambda i: (i, 0)),
        out_shape=jax.ShapeDtypeStruct((B, HP), jnp.float32),
    )(seq_hp, last_h, valid, maskf, etp, w2, w3, vp, nbp)


def _pad_body(x_ref, o_ref):
    i = pl.program_id(0)
    rowv = i * RB + lax.broadcasted_iota(jnp.int32, (RB, H), 0) < N
    x = jnp.where(rowv, x_ref[...], 0.0)
    o_ref[...] = jnp.concatenate([x, jnp.zeros((RB, HP - H), jnp.float32)],
                                 axis=1)


def _pad_stage(embedding):
    return pl.pallas_call(
        _pad_body,
        grid=(RT // RB,),
        in_specs=[pl.BlockSpec((RB, H), lambda i: (i, 0))],
        out_specs=pl.BlockSpec((RB, HP), lambda i: (i, 0)),
        out_shape=jax.ShapeDtypeStruct((RT, HP), jnp.float32),
    )(embedding)


def _logits_body(ma_ref, bt0_ref, bt1_ref, lab_ref, o_ref, mx_ref, sm_ref,
                 lg_ref):
    i = pl.program_id(0)
    bt = jnp.concatenate([bt0_ref[1:, :], bt1_ref[:1, :]], axis=0)
    lg = lax.dot_general(ma_ref[...], bt, (((1,), (1,)), ((), ())),
                         preferred_element_type=jnp.float32, precision=lax.Precision.HIGHEST)
    col = i * CB + lax.broadcasted_iota(jnp.int32, (B, CB), 1)
    okc = col < V
    lgm = jnp.where(okc, lg, -1e30)
    o_ref[...] = lg
    bm = jnp.max(lgm, axis=1, keepdims=True)
    bs = jnp.sum(jnp.exp(lgm - bm), axis=1, keepdims=True)
    labs = lab_ref[...]
    hit = jnp.where((col == labs) & okc, lg, 0.0)
    lab_part = jnp.sum(hit, axis=1, keepdims=True)

    @pl.when(i == 0)
    def _():
        mx_ref[...] = bm
        sm_ref[...] = bs
        lg_ref[...] = lab_part

    @pl.when(i > 0)
    def _():
        m_old = mx_ref[...]
        m_new = jnp.maximum(m_old, bm)
        sm_ref[...] = (sm_ref[...] * jnp.exp(m_old - m_new)
                       + bs * jnp.exp(bm - m_new))
        mx_ref[...] = m_new
        lg_ref[...] += lab_part


def _logits_stage(map_, e0p, labels):
    nblk = pl.cdiv(V, CB)
    last = pl.cdiv(RT, CB) - 1
    return pl.pallas_call(
        _logits_body,
        grid=(nblk,),
        in_specs=[
            pl.BlockSpec((B, HP), lambda i: (0, 0)),
            pl.BlockSpec((CB, HP), lambda i: (i, 0)),
            pl.BlockSpec((CB, HP), lambda i: (jnp.minimum(i + 1, last), 0)),
            pl.BlockSpec((B, 1), lambda i: (0, 0)),
        ],
        out_specs=[
            pl.BlockSpec((B, CB), lambda i: (0, i)),
            pl.BlockSpec((B, 1), lambda i: (0, 0)),
            pl.BlockSpec((B, 1), lambda i: (0, 0)),
            pl.BlockSpec((B, 1), lambda i: (0, 0)),
        ],
        out_shape=[
            jax.ShapeDtypeStruct((B, V), jnp.float32),
            jax.ShapeDtypeStruct((B, 1), jnp.float32),
            jax.ShapeDtypeStruct((B, 1), jnp.float32),
            jax.ShapeDtypeStruct((B, 1), jnp.float32),
        ],
    )(map_, e0p, e0p, labels)


_NC, _NS = 2, 16
_CH = 128
_EPAD = 811008
_ET = _EPAD // _NS
_NCHUNK = _ET // _CH
_TRW = RT // _NS
_DR = 112


def _sc_mesh():
    return plsc.VectorSubcoreMesh(core_axis_name="c", subcore_axis_name="s",
                                  num_cores=_NC, num_subcores=_NS)


_NB = 6
_NG = _NCHUNK // _NB


def _sc_scatter_body(ego_ref, src_ref, dst_ref, out_ref,
                     acc, si, di, gi, rows, didx,
                     semi, semg, sems, semd):
    c = lax.axis_index("c")
    s = lax.axis_index("s")
    iot = lax.iota(jnp.int32, 16)
    z16 = jnp.zeros((16,), jnp.float32)

    def zero_rows0():
        def zb(r, _):
            rows[0, r, pl.ds(0, 16)] = z16
            rows[0, r, pl.ds(16, 16)] = z16
            return 0
        lax.fori_loop(0, _DR, zb, 0)

    def fire_idx(jc, b):
        eb = s * _ET + jc * _CH
        pltpu.async_copy(src_ref.at[pl.ds(eb, _CH)], si.at[b], semi.at[b])
        pltpu.async_copy(dst_ref.at[pl.ds(eb, _CH)], di.at[b], semi.at[b])

    zsrc = rows.at[0, pl.ds(0, _DR)]
    for p in range(2):
        chunk = 2 * p + c

        if p == 0:
            for b in range(_NB):
                fire_idx(b, b)
            zero_rows0()

            def zacc(k, _):
                pltpu.sync_copy(zsrc, acc.at[pl.ds(s * _TRW + k * _DR, _DR)])
                return 0
            lax.fori_loop(0, _TRW // _DR, zacc, 0)
        plsc.subcore_barrier()

        def group(g, _):
            gds = []
            for b in range(_NB):
                pltpu.make_async_copy(src_ref.at[pl.ds(0, _CH)],
                                      si.at[b], semi.at[b]).wait()
                pltpu.make_async_copy(dst_ref.at[pl.ds(0, _CH)],
                                      di.at[b], semi.at[b]).wait()
                for gg in range(_CH // 16):
                    sl = pl.ds(gg * 16, 16)
                    gi[b, sl] = si[b, sl] * 4 + chunk
                gds.append(pltpu.async_copy(ego_ref.at[gi.at[b]],
                                            rows.at[b], semg.at[b]))
            sds = []
            for b in range(_NB):
                gds[b].wait()
                sds.append(pltpu.async_copy(rows.at[b], acc.at[di.at[b]],
                                            sems.at[b], add=True))
            for b in range(_NB):
                sds[b].wait()

                @pl.when(g < _NG - 1)
                def _():
                    fire_idx((g + 1) * _NB + b, b)
            return 0
        lax.fori_loop(0, _NG, group, 0)
        if p == 0:
            for b in range(_NB):
                fire_idx(b, b)
        plsc.subcore_barrier()

        if p == 0:
            zero_rows0()
        def dpair(k, _):
            for bb in range(2):
                kk = 2 * k + bb
                dsl = rows.at[1 + bb, pl.ds(0, _DR)]

                @pl.when(k > 0)
                def _():
                    pltpu.make_async_copy(dsl, out_ref.at[didx.at[bb]],
                                          semd.at[bb]).wait()
                rb = s * _TRW + kk * _DR
                pltpu.sync_copy(acc.at[pl.ds(rb, _DR)], dsl)
                if p == 0:
                    pltpu.sync_copy(zsrc, acc.at[pl.ds(rb, _DR)])
                for gg in range(_DR // 16):
                    didx[bb, pl.ds(gg * 16, 16)] = (4 * rb + 64 * gg
                                                    + 4 * iot + chunk)
                pltpu.async_copy(dsl, out_ref.at[didx.at[bb]], semd.at[bb])
            return 0
        lax.fori_loop(0, _TRW // _DR // 2, dpair, 0)
        for bb in range(2):
            pltpu.make_async_copy(rows.at[1 + bb, pl.ds(0, _DR)],
                                  out_ref.at[didx.at[bb]],
                                  semd.at[bb]).wait()


def _scatter_layer(ep, srcp, dstp):
    ego_v = ep.reshape(RT * 4, 32)
    run = functools.partial(
        pl.kernel,
        out_type=jax.ShapeDtypeStruct((RT * 4, 32), jnp.float32),
        mesh=_sc_mesh(),
        compiler_params=pltpu.CompilerParams(use_tc_tiling_on_sc=False),
        scratch_types=[
            pltpu.VMEM_SHARED((RT, 32), jnp.float32),
            pltpu.VMEM((_NB, _CH), jnp.int32),
            pltpu.VMEM((_NB, _CH), jnp.int32),
            pltpu.VMEM((_NB, _CH), jnp.int32),
            pltpu.VMEM((_NB, _CH, 32), jnp.float32),
            pltpu.VMEM((2, _DR), jnp.int32),
            pltpu.SemaphoreType.DMA((_NB,)),
            pltpu.SemaphoreType.DMA((_NB,)),
            pltpu.SemaphoreType.DMA((_NB,)),
            pltpu.SemaphoreType.DMA((2,)),
        ],
    )(_sc_scatter_body)
    return run(ego_v, srcp, dstp).reshape(RT, HP)


_GW = (B * T) // (_NC * _NS)


def _sc_gather_body(tab_ref, gi_ref, out_ref, i128, i16, r128, r16, sem):
    cc = lax.axis_index("c")
    ss = lax.axis_index("s")
    base = (ss * _NC + cc) * _GW
    for off in (0, 128, 256):
        pltpu.sync_copy(gi_ref.at[pl.ds(base + off, 128)], i128)
        pltpu.async_copy(tab_ref.at[i128], r128, sem).wait()
        pltpu.sync_copy(r128, out_ref.at[pl.ds(base + off, 128)])
    pltpu.sync_copy(gi_ref.at[pl.ds(base + 384, 16)], i16)
    pltpu.async_copy(tab_ref.at[i16], r16, sem).wait()
    pltpu.sync_copy(r16, out_ref.at[pl.ds(base + 384, 16)])


def _session_gather(finalp, gidx):
    run = functools.partial(
        pl.kernel,
        out_type=jax.ShapeDtypeStruct((B * T, HP), jnp.float32),
        mesh=_sc_mesh(),
        scratch_types=[
            pltpu.VMEM((128,), jnp.int32),
            pltpu.VMEM((16,), jnp.int32),
            pltpu.VMEM((128, HP), jnp.float32),
            pltpu.VMEM((16, HP), jnp.float32),
            pltpu.SemaphoreType.DMA,
        ],
    )(_sc_gather_body)
    return run(finalp, gidx)


def _pad_w(w):
    return jnp.pad(w, ((0, HP - w.shape[0]), (0, HP - w.shape[1])))


def _pad_b(b):
    return jnp.pad(b.reshape(1, -1), ((0, 0), (0, HP - b.shape[-1])))


@jax.jit
def kernel(edge_index, item, alias, mask, tar, embedding,
           W_gc_0, b_gc_0, W_gc_1, b_gc_1, W_concat, b_concat,
           nasr_w2, nasr_w3, nasr_v, nasr_b):
    npad = _EPAD - E
    srcp = jnp.concatenate([edge_index[0],
                            (jnp.arange(npad, dtype=jnp.int32) * 131) % N])
    dstp = jnp.concatenate([edge_index[1],
                            N + (jnp.arange(npad, dtype=jnp.int32) % (RT - N))])

    e0p = _pad_stage(embedding)
    w0p = _pad_w(W_gc_0)
    b0p = _pad_b(b_gc_0)
    w1p = _pad_w(W_gc_1)
    b1p = _pad_b(b_gc_1)
    wc0p = _pad_w(W_concat[:H])
    wc1p = _pad_w(W_concat[H:2 * H])
    wc2p = _pad_w(W_concat[2 * H:])
    bcp = _pad_b(b_concat)
    w2p = _pad_w(nasr_w2)
    w3p = _pad_w(nasr_w3)
    vpp = _pad_b(nasr_v[0])
    nbp = _pad_b(nasr_b)

    agg1p = _scatter_layer(e0p, srcp, dstp)
    e1p = _mm_act(agg1p, w0p, b0p)
    agg2p = _scatter_layer(e1p, srcp, dstp)
    finalp, ssq_e0 = _final_stage(agg2p, e0p, e1p, w1p, b1p,
                                  wc0p, wc1p, wc2p, bcp)

    gidx = jnp.take_along_axis(item, alias, axis=1)
    seq_hp = _session_gather(finalp, gidx.reshape(-1))
    last_h = seq_hp.reshape(B, T, HP)[:, T - 1]
    valid = (item != 0).astype(jnp.float32)
    maskf = mask.reshape(B * T, 1)
    etp = finalp[:T]

    map_ = _session_stage(seq_hp, last_h, valid, maskf, etp,
                          w2p, w3p, vpp, nbp)

    labels = jnp.clip(tar - 1, 0, N - 2).astype(jnp.int32).reshape(B, 1)
    logits, rmax, rsum, rlab = _logits_stage(map_, e0p, labels)

    lse = jnp.log(rsum[:, 0]) + rmax[:, 0]
    ce = jnp.mean(lse - rlab[:, 0])
    wsum = (jnp.sum(W_gc_0 ** 2) + jnp.sum(b_gc_0 ** 2)
            + jnp.sum(W_gc_1 ** 2) + jnp.sum(b_gc_1 ** 2)
            + jnp.sum(W_concat ** 2) + jnp.sum(b_concat ** 2)
            + jnp.sum(nasr_w2 ** 2) + jnp.sum(nasr_w3 ** 2)
            + jnp.sum(nasr_v ** 2) + jnp.sum(nasr_b ** 2))
    l2 = 0.5 * (ssq_e0[0, 0] + wsum) * L2C
    loss = ce + l2
    return (loss, logits)


# speedup vs baseline: 8.1299x; 1.0910x over previous
import functools

import jax
import jax.numpy as jnp
from jax import lax
from jax.experimental import pallas as pl
from jax.experimental.pallas import tpu as pltpu
from jax.experimental.pallas import tpu_sc as plsc

N = 50000
H = 100
HP = 128
B = 256
T = 50
E = 800000
V = N - 1
L2C = 1e-05

RT = 50176
RB = 3136
CB = 2048


def _leaky(x):
    return jnp.where(x >= 0, x, 0.2 * x)


def _mm_act_body(x_ref, w_ref, b_ref, o_ref):
    o_ref[...] = _leaky(
        jnp.dot(x_ref[...], w_ref[...], preferred_element_type=jnp.float32)
        + b_ref[...])


def _mm_act(x, w, b):
    return pl.pallas_call(
        _mm_act_body,
        grid=(RT // RB,),
        in_specs=[
            pl.BlockSpec((RB, HP), lambda i: (i, 0)),
            pl.BlockSpec((HP, HP), lambda i: (0, 0)),
            pl.BlockSpec((1, HP), lambda i: (0, 0)),
        ],
        out_specs=pl.BlockSpec((RB, HP), lambda i: (i, 0)),
        out_shape=jax.ShapeDtypeStruct((RT, HP), jnp.float32),
    )(x, w, b)


def _final_body(agg2_ref, e0_ref, e1_ref, w1_ref, b1_ref,
                wc0_ref, wc1_ref, wc2_ref, bc_ref, o_ref, ssq_ref):
    i = pl.program_id(0)
    e0 = e0_ref[...]
    e2 = _leaky(
        jnp.dot(agg2_ref[...], w1_ref[...], preferred_element_type=jnp.float32)
        + b1_ref[...])
    f = (jnp.dot(e0, wc0_ref[...], preferred_element_type=jnp.float32)
         + jnp.dot(e1_ref[...], wc1_ref[...], preferred_element_type=jnp.float32)
         + jnp.dot(e2, wc2_ref[...], preferred_element_type=jnp.float32)
         + bc_ref[...])
    o_ref[...] = _leaky(f)
    part = jnp.sum(e0 * e0).reshape(1, 1)

    @pl.when(i == 0)
    def _():
        ssq_ref[...] = part

    @pl.when(i > 0)
    def _():
        ssq_ref[...] += part


def _final_stage(agg2p, e0p, e1p, w1, b1, wc0, wc1, wc2, bc):
    return pl.pallas_call(
        _final_body,
        grid=(RT // RB,),
        in_specs=[
            pl.BlockSpec((RB, HP), lambda i: (i, 0)),
            pl.BlockSpec((RB, HP), lambda i: (i, 0)),
            pl.BlockSpec((RB, HP), lambda i: (i, 0)),
        ] + [pl.BlockSpec((HP, HP), lambda i: (0, 0))] * 1
          + [pl.BlockSpec((1, HP), lambda i: (0, 0))]
          + [pl.BlockSpec((HP, HP), lambda i: (0, 0))] * 3
          + [pl.BlockSpec((1, HP), lambda i: (0, 0))],
        out_specs=[
            pl.BlockSpec((RB, HP), lambda i: (i, 0)),
            pl.BlockSpec((1, 1), lambda i: (0, 0)),
        ],
        out_shape=[
            jax.ShapeDtypeStruct((RT, HP), jnp.float32),
            jax.ShapeDtypeStruct((1, 1), jnp.float32),
        ],
    )(agg2p, e0p, e1p, w1, b1, wc0, wc1, wc2, bc)


_SB = 32
_XB = _SB * T


def _session_body(x_ref, last_ref, valid_ref, maskf_ref, et_ref,
                  w2_ref, w3_ref, v_ref, nb_ref, ma_ref):
    x = x_ref[...]
    lasth = last_ref[...]
    ri = lax.broadcasted_iota(jnp.int32, (_XB, _SB), 0) // T
    ci = lax.broadcasted_iota(jnp.int32, (_XB, _SB), 1)
    sel = (ri == ci).astype(jnp.float32)
    lastb = jnp.dot(sel, lasth, preferred_element_type=jnp.float32, precision=lax.Precision.HIGHEST)
    seq = jnp.dot(x, w2_ref[...], preferred_element_type=jnp.float32)
    m = jax.nn.sigmoid(lastb + seq + nb_ref[...])
    coef = jnp.dot(m, v_ref[...].T, preferred_element_type=jnp.float32)
    coef = coef * maskf_ref[...]
    glb = lax.dot_general(sel, coef * x, (((0,), (0,)), ((), ())),
                          preferred_element_type=jnp.float32,
                          precision=lax.Precision.HIGHEST)
    vald = valid_ref[...]
    cnt = jnp.clip(jnp.sum(vald, axis=1, keepdims=True), 1.0, None)
    last_emb = jnp.dot(vald, et_ref[...], preferred_element_type=jnp.float32) / cnt
    gate = glb + jnp.dot(last_emb, w3_ref[...], preferred_element_type=jnp.float32)
    ma_ref[...] = gate * lasth


def _session_stage(seq_hp, last_h, valid, maskf, etp, w2, w3, vp, nbp):
    return pl.pallas_call(
        _session_body,
        grid=(B // _SB,),
        in_specs=[
            pl.BlockSpec((_XB, HP), lambda i: (i, 0)),
            pl.BlockSpec((_SB, HP), lambda i: (i, 0)),
            pl.BlockSpec((_SB, T), lambda i: (i, 0)),
            pl.BlockSpec((_XB, 1), lambda i: (i, 0)),
            pl.BlockSpec((T, HP), lambda i: (0, 0)),
            pl.BlockSpec((HP, HP), lambda i: (0, 0)),
            pl.BlockSpec((HP, HP), lambda i: (0, 0)),
            pl.BlockSpec((1, HP), lambda i: (0, 0)),
            pl.BlockSpec((1, HP), lambda i: (0, 0)),
        ],
        out_specs=pl.BlockSpec((_SB, HP), lambda i: (i, 0)),
        out_shape=jax.ShapeDtypeStruct((B, HP), jnp.float32),
    )(seq_hp, last_h, valid, maskf, etp, w2, w3, vp, nbp)


def _pad_body(x_ref, o_ref):
    i = pl.program_id(0)
    rowv = i * RB + lax.broadcasted_iota(jnp.int32, (RB, H), 0) < N
    x = jnp.where(rowv, x_ref[...], 0.0)
    o_ref[...] = jnp.concatenate([x, jnp.zeros((RB, HP - H), jnp.float32)],
                                 axis=1)


def _pad_stage(embedding):
    return pl.pallas_call(
        _pad_body,
        grid=(RT // RB,),
        in_specs=[pl.BlockSpec((RB, H), lambda i: (i, 0))],
        out_specs=pl.BlockSpec((RB, HP), lambda i: (i, 0)),
        out_shape=jax.ShapeDtypeStruct((RT, HP), jnp.float32),
    )(embedding)


def _logits_body(ma_ref, bt0_ref, bt1_ref, lab_ref, o_ref, mx_ref, sm_ref,
                 lg_ref):
    i = pl.program_id(0)
    bt = jnp.concatenate([bt0_ref[1:, :], bt1_ref[:1, :]], axis=0)
    lg = lax.dot_general(ma_ref[...], bt, (((1,), (1,)), ((), ())),
                         preferred_element_type=jnp.float32)
    col = i * CB + lax.broadcasted_iota(jnp.int32, (B, CB), 1)
    okc = col < V
    lgm = jnp.where(okc, lg, -1e30)
    o_ref[...] = lg
    bm = jnp.max(lgm, axis=1, keepdims=True)
    bs = jnp.sum(jnp.exp(lgm - bm), axis=1, keepdims=True)
    labs = lab_ref[...]
    hit = jnp.where((col == labs) & okc, lg, 0.0)
    lab_part = jnp.sum(hit, axis=1, keepdims=True)

    @pl.when(i == 0)
    def _():
        mx_ref[...] = bm
        sm_ref[...] = bs
        lg_ref[...] = lab_part

    @pl.when(i > 0)
    def _():
        m_old = mx_ref[...]
        m_new = jnp.maximum(m_old, bm)
        sm_ref[...] = (sm_ref[...] * jnp.exp(m_old - m_new)
                       + bs * jnp.exp(bm - m_new))
        mx_ref[...] = m_new
        lg_ref[...] += lab_part


def _logits_stage(map_, e0p, labels):
    nblk = pl.cdiv(V, CB)
    last = pl.cdiv(RT, CB) - 1
    return pl.pallas_call(
        _logits_body,
        grid=(nblk,),
        in_specs=[
            pl.BlockSpec((B, HP), lambda i: (0, 0)),
            pl.BlockSpec((CB, HP), lambda i: (i, 0)),
            pl.BlockSpec((CB, HP), lambda i: (jnp.minimum(i + 1, last), 0)),
            pl.BlockSpec((B, 1), lambda i: (0, 0)),
        ],
        out_specs=[
            pl.BlockSpec((B, CB), lambda i: (0, i)),
            pl.BlockSpec((B, 1), lambda i: (0, 0)),
            pl.BlockSpec((B, 1), lambda i: (0, 0)),
            pl.BlockSpec((B, 1), lambda i: (0, 0)),
        ],
        out_shape=[
            jax.ShapeDtypeStruct((B, V), jnp.float32),
            jax.ShapeDtypeStruct((B, 1), jnp.float32),
            jax.ShapeDtypeStruct((B, 1), jnp.float32),
            jax.ShapeDtypeStruct((B, 1), jnp.float32),
        ],
    )(map_, e0p, e0p, labels)


_NC, _NS = 2, 16
_CH = 128
_EPAD = 811008
_ET = _EPAD // _NS
_NCHUNK = _ET // _CH
_TRW = RT // _NS
_DR = 112


def _sc_mesh():
    return plsc.VectorSubcoreMesh(core_axis_name="c", subcore_axis_name="s",
                                  num_cores=_NC, num_subcores=_NS)


_NB = 6
_NG = _NCHUNK // _NB


def _sc_scatter_body(ego_ref, src_ref, dst_ref, out_ref,
                     acc, si, di, gi, rows, didx,
                     semi, semg, sems, semd):
    c = lax.axis_index("c")
    s = lax.axis_index("s")
    iot = lax.iota(jnp.int32, 16)
    z16 = jnp.zeros((16,), jnp.float32)

    def zero_rows0():
        def zb(r, _):
            rows[0, r, pl.ds(0, 16)] = z16
            rows[0, r, pl.ds(16, 16)] = z16
            return 0
        lax.fori_loop(0, _DR, zb, 0)

    def fire_idx(jc, b):
        eb = s * _ET + jc * _CH
        pltpu.async_copy(src_ref.at[pl.ds(eb, _CH)], si.at[b], semi.at[b])
        pltpu.async_copy(dst_ref.at[pl.ds(eb, _CH)], di.at[b], semi.at[b])

    zsrc = rows.at[0, pl.ds(0, _DR)]
    for p in range(2):
        chunk = 2 * p + c

        if p == 0:
            for b in range(_NB):
                fire_idx(b, b)
            zero_rows0()

            def zacc(k, _):
                pltpu.sync_copy(zsrc, acc.at[pl.ds(s * _TRW + k * _DR, _DR)])
                return 0
            lax.fori_loop(0, _TRW // _DR, zacc, 0)
        plsc.subcore_barrier()

        def group(g, _):
            gds = []
            for b in range(_NB):
                pltpu.make_async_copy(src_ref.at[pl.ds(0, _CH)],
                                      si.at[b], semi.at[b]).wait()
                pltpu.make_async_copy(dst_ref.at[pl.ds(0, _CH)],
                                      di.at[b], semi.at[b]).wait()
                for gg in range(_CH // 16):
                    sl = pl.ds(gg * 16, 16)
                    gi[b, sl] = si[b, sl] * 4 + chunk
                gds.append(pltpu.async_copy(ego_ref.at[gi.at[b]],
                                            rows.at[b], semg.at[b]))
            sds = []
            for b in range(_NB):
                gds[b].wait()
                sds.append(pltpu.async_copy(rows.at[b], acc.at[di.at[b]],
                                            sems.at[b], add=True))
            for b in range(_NB):
                sds[b].wait()

                @pl.when(g < _NG - 1)
                def _():
                    fire_idx((g + 1) * _NB + b, b)
            return 0
        lax.fori_loop(0, _NG, group, 0)
        if p == 0:
            for b in range(_NB):
                fire_idx(b, b)
        plsc.subcore_barrier()

        if p == 0:
            zero_rows0()
        def dpair(k, _):
            for bb in range(2):
                kk = 2 * k + bb
                dsl = rows.at[1 + bb, pl.ds(0, _DR)]

                @pl.when(k > 0)
                def _():
                    pltpu.make_async_copy(dsl, out_ref.at[didx.at[bb]],
                                          semd.at[bb]).wait()
                rb = s * _TRW + kk * _DR
                pltpu.sync_copy(acc.at[pl.ds(rb, _DR)], dsl)
                if p == 0:
                    pltpu.sync_copy(zsrc, acc.at[pl.ds(rb, _DR)])
                for gg in range(_DR // 16):
                    didx[bb, pl.ds(gg * 16, 16)] = (4 * rb + 64 * gg
                                                    + 4 * iot + chunk)
                pltpu.async_copy(dsl, out_ref.at[didx.at[bb]], semd.at[bb])
            return 0
        lax.fori_loop(0, _TRW // _DR // 2, dpair, 0)
        for bb in range(2):
            pltpu.make_async_copy(rows.at[1 + bb, pl.ds(0, _DR)],
                                  out_ref.at[didx.at[bb]],
                                  semd.at[bb]).wait()


def _scatter_layer(ep, srcp, dstp):
    ego_v = ep.reshape(RT * 4, 32)
    run = functools.partial(
        pl.kernel,
        out_type=jax.ShapeDtypeStruct((RT * 4, 32), jnp.float32),
        mesh=_sc_mesh(),
        compiler_params=pltpu.CompilerParams(use_tc_tiling_on_sc=False),
        scratch_types=[
            pltpu.VMEM_SHARED((RT, 32), jnp.float32),
            pltpu.VMEM((_NB, _CH), jnp.int32),
            pltpu.VMEM((_NB, _CH), jnp.int32),
            pltpu.VMEM((_NB, _CH), jnp.int32),
            pltpu.VMEM((_NB, _CH, 32), jnp.float32),
            pltpu.VMEM((2, _DR), jnp.int32),
            pltpu.SemaphoreType.DMA((_NB,)),
            pltpu.SemaphoreType.DMA((_NB,)),
            pltpu.SemaphoreType.DMA((_NB,)),
            pltpu.SemaphoreType.DMA((2,)),
        ],
    )(_sc_scatter_body)
    return run(ego_v, srcp, dstp).reshape(RT, HP)


_GW = (B * T) // (_NC * _NS)


def _sc_gather_body(tab_ref, gi_ref, out_ref, i128, i16, r128, r16, sem):
    cc = lax.axis_index("c")
    ss = lax.axis_index("s")
    base = (ss * _NC + cc) * _GW
    for off in (0, 128, 256):
        pltpu.sync_copy(gi_ref.at[pl.ds(base + off, 128)], i128)
        pltpu.async_copy(tab_ref.at[i128], r128, sem).wait()
        pltpu.sync_copy(r128, out_ref.at[pl.ds(base + off, 128)])
    pltpu.sync_copy(gi_ref.at[pl.ds(base + 384, 16)], i16)
    pltpu.async_copy(tab_ref.at[i16], r16, sem).wait()
    pltpu.sync_copy(r16, out_ref.at[pl.ds(base + 384, 16)])


def _session_gather(finalp, gidx):
    run = functools.partial(
        pl.kernel,
        out_type=jax.ShapeDtypeStruct((B * T, HP), jnp.float32),
        mesh=_sc_mesh(),
        scratch_types=[
            pltpu.VMEM((128,), jnp.int32),
            pltpu.VMEM((16,), jnp.int32),
            pltpu.VMEM((128, HP), jnp.float32),
            pltpu.VMEM((16, HP), jnp.float32),
            pltpu.SemaphoreType.DMA,
        ],
    )(_sc_gather_body)
    return run(finalp, gidx)


def _pad_w(w):
    return jnp.pad(w, ((0, HP - w.shape[0]), (0, HP - w.shape[1])))


def _pad_b(b):
    return jnp.pad(b.reshape(1, -1), ((0, 0), (0, HP - b.shape[-1])))


@jax.jit
def kernel(edge_index, item, alias, mask, tar, embedding,
           W_gc_0, b_gc_0, W_gc_1, b_gc_1, W_concat, b_concat,
           nasr_w2, nasr_w3, nasr_v, nasr_b):
    npad = _EPAD - E
    srcp = jnp.concatenate([edge_index[0],
                            (jnp.arange(npad, dtype=jnp.int32) * 131) % N])
    dstp = jnp.concatenate([edge_index[1],
                            N + (jnp.arange(npad, dtype=jnp.int32) % (RT - N))])

    e0p = _pad_stage(embedding)
    w0p = _pad_w(W_gc_0)
    b0p = _pad_b(b_gc_0)
    w1p = _pad_w(W_gc_1)
    b1p = _pad_b(b_gc_1)
    wc0p = _pad_w(W_concat[:H])
    wc1p = _pad_w(W_concat[H:2 * H])
    wc2p = _pad_w(W_concat[2 * H:])
    bcp = _pad_b(b_concat)
    w2p = _pad_w(nasr_w2)
    w3p = _pad_w(nasr_w3)
    vpp = _pad_b(nasr_v[0])
    nbp = _pad_b(nasr_b)

    agg1p = _scatter_layer(e0p, srcp, dstp)
    e1p = _mm_act(agg1p, w0p, b0p)
    agg2p = _scatter_layer(e1p, srcp, dstp)
    finalp, ssq_e0 = _final_stage(agg2p, e0p, e1p, w1p, b1p,
                                  wc0p, wc1p, wc2p, bcp)

    gidx = jnp.take_along_axis(item, alias, axis=1)
    seq_hp = _session_gather(finalp, gidx.reshape(-1))
    last_h = seq_hp.reshape(B, T, HP)[:, T - 1]
    valid = (item != 0).astype(jnp.float32)
    maskf = mask.reshape(B * T, 1)
    etp = finalp[:T]

    map_ = _session_stage(seq_hp, last_h, valid, maskf, etp,
                          w2p, w3p, vpp, nbp)

    labels = jnp.clip(tar - 1, 0, N - 2).astype(jnp.int32).reshape(B, 1)
    logits, rmax, rsum, rlab = _logits_stage(map_, e0p, labels)

    lse = jnp.log(rsum[:, 0]) + rmax[:, 0]
    ce = jnp.mean(lse - rlab[:, 0])
    wsum = (jnp.sum(W_gc_0 ** 2) + jnp.sum(b_gc_0 ** 2)
            + jnp.sum(W_gc_1 ** 2) + jnp.sum(b_gc_1 ** 2)
            + jnp.sum(W_concat ** 2) + jnp.sum(b_concat ** 2)
            + jnp.sum(nasr_w2 ** 2) + jnp.sum(nasr_w3 ** 2)
            + jnp.sum(nasr_v ** 2) + jnp.sum(nasr_b ** 2))
    l2 = 0.5 * (ssq_e0[0, 0] + wsum) * L2C
    loss = ce + l2
    return (loss, logits)
